# baseline (speedup 1.0000x reference)
import functools

import jax
import jax.numpy as jnp
from jax import lax
from jax.experimental import pallas as pl
from jax.experimental.pallas import tpu as pltpu
from jax.experimental.pallas import tpu_sc as plsc

N_NODES = 10000
N_EDGES = 160000
HID = 256
HH = 128
ETP = 512
NC, NS, L = 2, 16, 16
NW = NC * NS

EPW = N_EDGES // NS
CH = 40
NCH = EPW // CH
NPW = N_NODES // NS
ERW = ETP // NS

_f32 = jnp.float32
@functools.lru_cache(maxsize=None)
def _mesh():
    return plsc.VectorSubcoreMesh(core_axis_name="c", subcore_axis_name="s")


def _sigmoid(z):
    return 1.0 / (1.0 + jnp.exp(-z))


_NR8 = 624
_NREM = N_NODES - NS * _NR8


def _acc_rows_io(src_at, dst_at, sid):
    pltpu.sync_copy(src_at(pl.ds(sid * _NR8, _NR8)),
                    dst_at(pl.ds(sid * _NR8, _NR8)))

    @pl.when(sid == NS - 1)
    def _():
        pltpu.sync_copy(src_at(pl.ds(NS * _NR8, _NREM)),
                        dst_at(pl.ds(NS * _NR8, _NREM)))


NID_PAD = 10240
_GPW = NID_PAD // NW


@functools.lru_cache(maxsize=None)
def _gather_h0_kernel():
    @functools.partial(
        pl.kernel,
        out_type=jax.ShapeDtypeStruct((NID_PAD, HID), _f32),
        mesh=_mesh(),
        scratch_types=[
            pltpu.VMEM((_GPW,), jnp.int32),
            pltpu.VMEM((_GPW, HID), _f32),
            pltpu.SemaphoreType.DMA,
        ],
    )
    def _gather_h0(table, nid, out, idx_v, rows_v, sem):
        wid = lax.axis_index("s") * NC + lax.axis_index("c")
        base = wid * _GPW
        pltpu.sync_copy(nid.at[pl.ds(base, _GPW)], idx_v)
        pltpu.async_copy(table.at[idx_v], rows_v, sem).wait()
        pltpu.sync_copy(rows_v, out.at[pl.ds(base, _GPW)])

    return _gather_h0


@functools.lru_cache(maxsize=None)
def _make_edge_pass_a(with_x):
    n_x = 2 if with_x else 0
    scratch = [
        pltpu.VMEM((CH,), jnp.int32),
        pltpu.VMEM((CH,), jnp.int32),
        pltpu.VMEM((CH,), jnp.int32),
        pltpu.VMEM((CH, HID), _f32),
        pltpu.VMEM((CH, HH), _f32),
        pltpu.VMEM((CH, HH), _f32),
        pltpu.VMEM((CH, HH), _f32),
        pltpu.VMEM((CH, HH), _f32),
        pltpu.VMEM((CH, HH), _f32),
        pltpu.VMEM((2, HH), _f32),
        pltpu.VMEM_SHARED((N_NODES, HH), _f32),
        pltpu.SemaphoreType.DMA,
        pltpu.SemaphoreType.DMA,
        pltpu.SemaphoreType.DMA,
        pltpu.SemaphoreType.DMA,
    ]
    out_type = [
        jax.ShapeDtypeStruct((N_EDGES, HH), _f32),
        jax.ShapeDtypeStruct((N_EDGES, HH), _f32),
        jax.ShapeDtypeStruct((N_NODES, HH), _f32),
        jax.ShapeDtypeStruct((N_NODES, HH), _f32),
        jax.ShapeDtypeStruct((NS, 2, HH), _f32),
        jax.ShapeDtypeStruct((NS, 2, HH), _f32),
    ]

    @functools.partial(pl.kernel, out_type=out_type, mesh=_mesh(),
                       scratch_types=scratch)
    def edge_a(*refs):
        (src_h, dst_h, et_h, sT0, sT1, dT0, dT1, cT0, cT1) = refs[:9]
        xs = refs[9:9 + n_x]
        zeros_h = refs[9 + n_x]
        (z0_o, z1_o, n0_o, n1_o, st0_o, st1_o) = refs[10 + n_x:16 + n_x]
        (src_i, dst_i, et_i, srows, drows, crows, xrows, z_v, sb_v,
         stats_v, acc_sh, sem0, sem1, sem2, sem3) = refs[16 + n_x:]
        cid = lax.axis_index("c")
        sid = lax.axis_index("s")

        def run(sT, dT, cT, xT, z_o, n_o, st_o):
            _acc_rows_io(lambda s: zeros_h.at[s], lambda s: acc_sh.at[s], sid)
            plsc.subcore_barrier()

            def chunk(i, carry):
                base = sid * EPW + i * CH
                pltpu.sync_copy(src_h.at[pl.ds(base, CH)], src_i)
                pltpu.sync_copy(dst_h.at[pl.ds(base, CH)], dst_i)
                pltpu.sync_copy(et_h.at[pl.ds(base, CH)], et_i)
                d1 = pltpu.async_copy(sT.at[src_i], srows, sem0)
                d2 = pltpu.async_copy(dT.at[dst_i], drows, sem1)
                d3 = pltpu.async_copy(cT.at[et_i], crows, sem2)
                if with_x:
                    d4 = pltpu.async_copy(xT.at[pl.ds(base, CH)], xrows, sem3)
                d1.wait()
                d2.wait()
                d3.wait()
                if with_x:
                    d4.wait()

                def edge(j, car):
                    new = []
                    for r in range(8):
                        sl = pl.ds(r * L, L)
                        z = srows[j, sl] + drows[j, sl] + crows[j, sl]
                        if with_x:
                            z = z + xrows[j, sl]
                        sig = _sigmoid(z)
                        z_v[j, sl] = z
                        sb_v[j, sl] = sig * srows[j, pl.ds(HH + r * L, L)]
                        new.append(car[2 * r] + z)
                        new.append(car[2 * r + 1] + z * z)
                    return tuple(new)

                carry = lax.fori_loop(0, CH, edge, carry)
                pltpu.sync_copy(z_v, z_o.at[pl.ds(base, CH)])
                pltpu.sync_copy(sb_v, acc_sh.at[dst_i], add=True)
                return carry

            zero = jnp.zeros((L,), _f32)
            carry = lax.fori_loop(0, NCH, chunk, tuple(zero for _ in range(16)))
            for r in range(8):
                stats_v[0, pl.ds(r * L, L)] = carry[2 * r]
                stats_v[1, pl.ds(r * L, L)] = carry[2 * r + 1]
            pltpu.sync_copy(stats_v, st_o.at[sid])
            plsc.subcore_barrier()
            _acc_rows_io(lambda s: acc_sh.at[s], lambda s: n_o.at[s], sid)

        x0 = xs[0] if with_x else None
        x1 = xs[1] if with_x else None

        @pl.when(cid == 0)
        def _():
            run(sT0, dT0, cT0, x0, z0_o, n0_o, st0_o)

        @pl.when(cid == 1)
        def _():
            run(sT1, dT1, cT1, x1, z1_o, n1_o, st1_o)

    return edge_a


@functools.lru_cache(maxsize=None)
def _edge_pass_b_kernel():
  @functools.partial(
    pl.kernel,
    out_type=[
        jax.ShapeDtypeStruct((N_NODES, HH), _f32),
        jax.ShapeDtypeStruct((N_NODES, HH), _f32),
    ],
    mesh=_mesh(),
    scratch_types=[
        pltpu.VMEM((CH,), jnp.int32),
        pltpu.VMEM((CH, HH), _f32),
        pltpu.VMEM((CH, HH), _f32),
        pltpu.VMEM_SHARED((N_NODES, HH), _f32),
        pltpu.SemaphoreType.DMA,
    ],
  )
  def _edge_pass_b(dst_h, zT0, zT1, zeros_h, d0_o, d1_o,
                 dst_i, zrows, sg_v, acc_sh, semz):
    cid = lax.axis_index("c")
    sid = lax.axis_index("s")

    def run(zT, d_o):
        _acc_rows_io(lambda s: zeros_h.at[s], lambda s: acc_sh.at[s], sid)
        plsc.subcore_barrier()

        def chunk(i, _):
            base = sid * EPW + i * CH
            pltpu.sync_copy(dst_h.at[pl.ds(base, CH)], dst_i)
            pltpu.async_copy(zT.at[pl.ds(base, CH)], zrows, semz).wait()

            def edge(j, __):
                for r in range(8):
                    sl = pl.ds(r * L, L)
                    sg_v[j, sl] = _sigmoid(zrows[j, sl])
                return 0

            lax.fori_loop(0, CH, edge, 0)
            pltpu.sync_copy(sg_v, acc_sh.at[dst_i], add=True)
            return 0

        lax.fori_loop(0, NCH, chunk, 0)
        plsc.subcore_barrier()
        _acc_rows_io(lambda s: acc_sh.at[s], lambda s: d_o.at[s], sid)

    @pl.when(cid == 0)
    def _():
        run(zT0, d0_o)

    @pl.when(cid == 1)
    def _():
        run(zT1, d1_o)

  return _edge_pass_b


@functools.lru_cache(maxsize=None)
def _rel_pass_kernel():
  @functools.partial(
    pl.kernel,
    out_type=[
        jax.ShapeDtypeStruct((ETP, HH), _f32),
        jax.ShapeDtypeStruct((ETP, HH), _f32),
        jax.ShapeDtypeStruct((ETP, HH), _f32),
    ],
    mesh=_mesh(),
    scratch_types=[
        pltpu.VMEM((CH,), jnp.int32),
        pltpu.VMEM((CH, HH), _f32),
        pltpu.VMEM((CH, HH), _f32),
        pltpu.VMEM((CH, HH), _f32),
        pltpu.VMEM((CH, HH), _f32),
        pltpu.VMEM((2, HH), _f32),
        pltpu.VMEM((2, HH), _f32),
        pltpu.VMEM_SHARED((ETP, HH), _f32),
        pltpu.VMEM_SHARED((ETP, HH), _f32),
        pltpu.SemaphoreType.DMA,
        pltpu.SemaphoreType.DMA,
    ],
  )
  def _rel_pass(et_h, z0T0, z0T1, z1T0, z1T1, b00, b01, b10, b11,
              zeros_e, rs0_o, rs1_o, cnt_o,
              et_i, z0rows, z1rows, r_v, ones_v, bn0_v, bn1_v,
              rsum_sh, cnt_sh, sem0, sem1):
    cid = lax.axis_index("c")
    sid = lax.axis_index("s")

    def run(z0T, z1T, b0, b1, rs_o, count):
        pltpu.sync_copy(zeros_e.at[pl.ds(sid * ERW, ERW)],
                        rsum_sh.at[pl.ds(sid * ERW, ERW)])
        if count:
            pltpu.sync_copy(zeros_e.at[pl.ds(sid * ERW, ERW)],
                            cnt_sh.at[pl.ds(sid * ERW, ERW)])
        pltpu.sync_copy(b0, bn0_v)
        pltpu.sync_copy(b1, bn1_v)

        def fill(j, _):
            for r in range(8):
                ones_v[j, pl.ds(r * L, L)] = jnp.full((L,), 1.0, _f32)
            return 0

        if count:
            lax.fori_loop(0, CH, fill, 0)
        plsc.subcore_barrier()

        def chunk(i, _):
            base = sid * EPW + i * CH
            pltpu.sync_copy(et_h.at[pl.ds(base, CH)], et_i)
            d0 = pltpu.async_copy(z0T.at[pl.ds(base, CH)], z0rows, sem0)
            d1 = pltpu.async_copy(z1T.at[pl.ds(base, CH)], z1rows, sem1)
            d0.wait()
            d1.wait()

            def edge(j, __):
                for r in range(8):
                    sl = pl.ds(r * L, L)
                    a0 = bn0_v[0, sl]
                    c0 = bn0_v[1, sl]
                    a1 = bn1_v[0, sl]
                    c1 = bn1_v[1, sl]
                    r0 = jnp.maximum(a0 * z0rows[j, sl] + c0, 0.0)
                    r1 = jnp.maximum(a1 * z1rows[j, sl] + c1, 0.0)
                    r_v[j, sl] = r0 + r1
                return 0

            lax.fori_loop(0, CH, edge, 0)
            pltpu.sync_copy(r_v, rsum_sh.at[et_i], add=True)
            if count:
                pltpu.sync_copy(ones_v, cnt_sh.at[et_i], add=True)
            return 0

        lax.fori_loop(0, NCH, chunk, 0)
        plsc.subcore_barrier()
        pltpu.sync_copy(rsum_sh.at[pl.ds(sid * ERW, ERW)],
                        rs_o.at[pl.ds(sid * ERW, ERW)])
        if count:
            pltpu.sync_copy(cnt_sh.at[pl.ds(sid * ERW, ERW)],
                            cnt_o.at[pl.ds(sid * ERW, ERW)])

    @pl.when(cid == 0)
    def _():
        run(z0T0, z1T0, b00, b10, rs0_o, True)

    @pl.when(cid == 1)
    def _():
        run(z0T1, z1T1, b01, b11, rs1_o, False)

  return _rel_pass


_MB = 1000


def _dense_body(h_ref, w_ref, b_ref, a_ref, d0_ref, d1_ref, s0_ref, s1_ref):
    y = jnp.dot(h_ref[...], w_ref[...], preferred_element_type=_f32)
    y = y + b_ref[...]
    a_ref[...] = y[:, 0:256]
    d0_ref[...] = y[:, 256:384]
    d1_ref[...] = y[:, 384:512]
    s0_ref[...] = jnp.concatenate([y[:, 512:640], y[:, 768:896]], axis=1)
    s1_ref[...] = jnp.concatenate([y[:, 640:768], y[:, 896:1024]], axis=1)


def _dense(h, wcat, bcat):
    n = h.shape[0]
    grid = (n // _MB,)
    return pl.pallas_call(
        _dense_body,
        grid=grid,
        in_specs=[
            pl.BlockSpec((_MB, HID), lambda i: (i, 0)),
            pl.BlockSpec((HID, 4 * HID), lambda i: (0, 0)),
            pl.BlockSpec((1, 4 * HID), lambda i: (0, 0)),
        ],
        out_specs=[
            pl.BlockSpec((_MB, HID), lambda i: (i, 0)),
            pl.BlockSpec((_MB, HH), lambda i: (i, 0)),
            pl.BlockSpec((_MB, HH), lambda i: (i, 0)),
            pl.BlockSpec((_MB, HID), lambda i: (i, 0)),
            pl.BlockSpec((_MB, HID), lambda i: (i, 0)),
        ],
        out_shape=[
            jax.ShapeDtypeStruct((n, HID), _f32),
            jax.ShapeDtypeStruct((n, HH), _f32),
            jax.ShapeDtypeStruct((n, HH), _f32),
            jax.ShapeDtypeStruct((n, HID), _f32),
            jax.ShapeDtypeStruct((n, HID), _f32),
        ],
    )(h, wcat, bcat)


def _cet_body(e_ref, w0_ref, b0_ref, w1_ref, b1_ref, c00, c01, c10, c11):
    y0 = jnp.dot(e_ref[...], w0_ref[...], preferred_element_type=_f32) + b0_ref[...]
    y1 = jnp.dot(e_ref[...], w1_ref[...], preferred_element_type=_f32) + b1_ref[...]
    c00[...] = y0[:, :HH]
    c01[...] = y0[:, HH:]
    c10[...] = y1[:, :HH]
    c11[...] = y1[:, HH:]


def _cet(emb_e_pad, cw0, cb0, cw1, cb1):
    return pl.pallas_call(
        _cet_body,
        out_shape=[jax.ShapeDtypeStruct((ETP, HH), _f32)] * 4,
    )(emb_e_pad, cw0, cb0.reshape(1, HID), cw1, cb1.reshape(1, HID))


def _node_body(with_stats, *refs):
    if with_stats:
        (h_ref, a_ref, n0, n1, d0, d1, g_ref, b_ref, st0, st1,
         ge_ref, be_ref, out_ref, bn0_ref, bn1_ref) = refs
    else:
        (h_ref, a_ref, n0, n1, d0, d1, g_ref, b_ref, out_ref) = refs
    num = jnp.concatenate([n0[...], n1[...]], axis=1)
    den = jnp.concatenate([d0[...], d1[...]], axis=1)
    x = a_ref[...] + num / (den + 1e-6)
    mu = jnp.mean(x, axis=0, keepdims=True)
    var = jnp.mean((x - mu) ** 2, axis=0, keepdims=True)
    y = g_ref[...] * (x - mu) / jnp.sqrt(var + 1e-5) + b_ref[...]
    out_ref[...] = h_ref[...] + jnp.maximum(y, 0.0)
    if with_stats:
        s0 = jnp.sum(st0[...], axis=0)
        s1 = jnp.sum(st1[...], axis=0)
        for s, ref, lo in ((s0, bn0_ref, 0), (s1, bn1_ref, HH)):
            mean = s[0:1] / N_EDGES
            v = s[1:2] / N_EDGES - mean * mean
            aa = ge_ref[:, lo:lo + HH] / jnp.sqrt(v + 1e-5)
            cc = be_ref[:, lo:lo + HH] - aa * mean
            ref[...] = jnp.concatenate([aa, cc], axis=0)


def _node_update(h_in, a, n0, n1, d0, d1, g, b, stats=None, ge=None, be=None):
    n = h_in.shape[0]
    if stats is None:
        out_shape = jax.ShapeDtypeStruct((n, HID), _f32)
        args = (h_in, a, n0, n1, d0, d1, g.reshape(1, HID), b.reshape(1, HID))
        body = functools.partial(_node_body, False)
    else:
        out_shape = [
            jax.ShapeDtypeStruct((n, HID), _f32),
            jax.ShapeDtypeStruct((2, HH), _f32),
            jax.ShapeDtypeStruct((2, HH), _f32),
        ]
        args = (h_in, a, n0, n1, d0, d1, g.reshape(1, HID), b.reshape(1, HID),
                stats[0], stats[1], ge.reshape(1, HID), be.reshape(1, HID))
        body = functools.partial(_node_body, True)
    return pl.pallas_call(
        body,
        out_shape=out_shape,
        compiler_params=pltpu.CompilerParams(
            vmem_limit_bytes=110 * 1024 * 1024),
    )(*args)


_CMB = 1600


def _cm_body(z0_ref, z1_ref, st0_ref, st1_ref, cw_ref, ge_ref, be_ref,
             x0_ref, x1_ref, bn0_ref, bn1_ref):
    s0 = jnp.sum(st0_ref[...], axis=0)
    s1 = jnp.sum(st1_ref[...], axis=0)
    mean = jnp.concatenate([s0[0:1], s1[0:1]], axis=1) / N_EDGES
    var = jnp.concatenate([s0[1:2], s1[1:2]], axis=1) / N_EDGES - mean * mean
    a = ge_ref[...] / jnp.sqrt(var + 1e-5)
    c = be_ref[...] - a * mean
    z = jnp.concatenate([z0_ref[...], z1_ref[...]], axis=1)
    r = jnp.maximum(a * z + c, 0.0)
    x = jnp.dot(r, cw_ref[...], preferred_element_type=_f32)
    x0_ref[...] = x[:, :HH]
    x1_ref[...] = x[:, HH:]
    bn0_ref[...] = jnp.concatenate([a[:, :HH], c[:, :HH]], axis=0)
    bn1_ref[...] = jnp.concatenate([a[:, HH:], c[:, HH:]], axis=0)


def _cmatmul(z0, z1, st0, st1, cw, ge, be):
    grid = (N_EDGES // _CMB,)
    return pl.pallas_call(
        _cm_body,
        grid=grid,
        in_specs=[
            pl.BlockSpec((_CMB, HH), lambda i: (i, 0)),
            pl.BlockSpec((_CMB, HH), lambda i: (i, 0)),
            pl.BlockSpec((NS, 2, HH), lambda i: (0, 0, 0)),
            pl.BlockSpec((NS, 2, HH), lambda i: (0, 0, 0)),
            pl.BlockSpec((HID, HID), lambda i: (0, 0)),
            pl.BlockSpec((1, HID), lambda i: (0, 0)),
            pl.BlockSpec((1, HID), lambda i: (0, 0)),
        ],
        out_specs=[
            pl.BlockSpec((_CMB, HH), lambda i: (i, 0)),
            pl.BlockSpec((_CMB, HH), lambda i: (i, 0)),
            pl.BlockSpec((2, HH), lambda i: (0, 0)),
            pl.BlockSpec((2, HH), lambda i: (0, 0)),
        ],
        out_shape=[
            jax.ShapeDtypeStruct((N_EDGES, HH), _f32),
            jax.ShapeDtypeStruct((N_EDGES, HH), _f32),
            jax.ShapeDtypeStruct((2, HH), _f32),
            jax.ShapeDtypeStruct((2, HH), _f32),
        ],
    )(z0, z1, st0, st1, cw, ge.reshape(1, HID), be.reshape(1, HID))


def _rel_body(r0_ref, r1_ref, cnt_ref, emb_ref, out_ref):
    cnt = cnt_ref[:, 0:1]
    rsum = jnp.concatenate([r0_ref[...], r1_ref[...]], axis=1)
    total = cnt * emb_ref[...] + rsum
    out_ref[...] = total / jnp.maximum(cnt, 1.0)


def _rel_means(r0, r1, cnt, emb_e_pad):
    return pl.pallas_call(
        _rel_body,
        out_shape=jax.ShapeDtypeStruct((ETP, HID), _f32),
    )(r0, r1, cnt, emb_e_pad)


def kernel(node_id, edge_index, edge_type, p, emb_h, emb_e,
           Aw, Ab, Bw, Bb, Cw, Cb, Dw, Db, Ew, Eb, gh, bh, ge, be):
    src = edge_index[0].astype(jnp.int32)
    dst = edge_index[1].astype(jnp.int32)
    et = edge_type.astype(jnp.int32)

    nid_pad = jnp.concatenate(
        [node_id.astype(jnp.int32), jnp.zeros((NID_PAD - N_NODES,), jnp.int32)])
    emb_e_pad = jnp.concatenate(
        [emb_e, jnp.zeros((ETP - emb_e.shape[0], HID), _f32)], axis=0)

    zeros_n = jnp.zeros((N_NODES, HH), _f32)
    zeros_e = jnp.zeros((ETP, HH), _f32)

    h0 = _gather_h0_kernel()(emb_h, nid_pad)[:N_NODES]

    cet = _cet(emb_e_pad, Cw[0], Cb[0], Cw[1], Cb[1])

    def wcat(l):
        w = jnp.concatenate([Aw[l], Ew[l], Dw[l], Bw[l]], axis=1)
        bb = jnp.concatenate([Ab[l], Eb[l], Db[l], Bb[l]]).reshape(1, 4 * HID)
        return w, bb

    w0, b0 = wcat(0)
    a0, e00, e01, s00, s01 = _dense(h0, w0, b0)
    z00, z01, n00, n01, st00, st01 = _make_edge_pass_a(False)(
        src, dst, et, s00, s01, e00, e01, cet[0], cet[1], zeros_n)
    d00, d01 = _edge_pass_b_kernel()(dst, z00, z01, zeros_n)
    h1 = _node_update(h0, a0, n00, n01, d00, d01, gh[0], bh[0])

    w1, b1 = wcat(1)
    a1, e10, e11, s10, s11 = _dense(h1, w1, b1)
    x0, x1, bn00, bn01 = _cmatmul(z00, z01, st00, st01, Cw[1], ge[0], be[0])
    z10, z11, n10, n11, st10, st11 = _make_edge_pass_a(True)(
        src, dst, et, s10, s11, e10, e11, cet[2], cet[3], x0, x1, zeros_n)
    d10, d11 = _edge_pass_b_kernel()(dst, z10, z11, zeros_n)
    h2, bn10, bn11 = _node_update(h1, a1, n10, n11, d10, d11, gh[1], bh[1],
                                  stats=(st10, st11), ge=ge[1], be=be[1])

    rs0, rs1, cnt = _rel_pass_kernel()(et, z00, z01, z10, z11,
                              bn00, bn01, bn10, bn11, zeros_e)
    rel = _rel_means(rs0, rs1, cnt, emb_e_pad)[:emb_e.shape[0]]
    return (h2, rel)


# speedup vs baseline: 1.2753x; 1.2753x over previous
import functools

import jax
import jax.numpy as jnp
from jax import lax
from jax.experimental import pallas as pl
from jax.experimental.pallas import tpu as pltpu
from jax.experimental.pallas import tpu_sc as plsc

N_NODES = 10000
N_EDGES = 160000
HID = 256
HH = 128
ETP = 512
NC, NS, L = 2, 16, 16
NW = NC * NS

EPW = N_EDGES // NS
CH = 40
NCH = EPW // CH
NPW = N_NODES // NS
ERW = ETP // NS

_f32 = jnp.float32
@functools.lru_cache(maxsize=None)
def _mesh():
    return plsc.VectorSubcoreMesh(core_axis_name="c", subcore_axis_name="s")


def _sigmoid(z):
    return 1.0 / (1.0 + jnp.exp(-z))


_NR8 = 624
_NREM = N_NODES - NS * _NR8


def _acc_rows_io(src_at, dst_at, sid):
    pltpu.sync_copy(src_at(pl.ds(sid * _NR8, _NR8)),
                    dst_at(pl.ds(sid * _NR8, _NR8)))

    @pl.when(sid == NS - 1)
    def _():
        pltpu.sync_copy(src_at(pl.ds(NS * _NR8, _NREM)),
                        dst_at(pl.ds(NS * _NR8, _NREM)))


NID_PAD = 10240
_GPW = NID_PAD // NW


@functools.lru_cache(maxsize=None)
def _gather_h0_kernel():
    @functools.partial(
        pl.kernel,
        out_type=jax.ShapeDtypeStruct((NID_PAD, HID), _f32),
        mesh=_mesh(),
        scratch_types=[
            pltpu.VMEM((_GPW,), jnp.int32),
            pltpu.VMEM((_GPW, HID), _f32),
            pltpu.SemaphoreType.DMA,
        ],
    )
    def _gather_h0(table, nid, out, idx_v, rows_v, sem):
        wid = lax.axis_index("s") * NC + lax.axis_index("c")
        base = wid * _GPW
        pltpu.sync_copy(nid.at[pl.ds(base, _GPW)], idx_v)
        pltpu.async_copy(table.at[idx_v], rows_v, sem).wait()
        pltpu.sync_copy(rows_v, out.at[pl.ds(base, _GPW)])

    return _gather_h0


@functools.lru_cache(maxsize=None)
def _make_edge_pass_a(with_x):
    n_x = 2 if with_x else 0
    scratch = [
        pltpu.VMEM((2, CH), jnp.int32),
        pltpu.VMEM((2, CH), jnp.int32),
        pltpu.VMEM((2, CH), jnp.int32),
        pltpu.VMEM((CH, HID), _f32),
        pltpu.VMEM((CH, HH), _f32),
        pltpu.VMEM((CH, HH), _f32),
        pltpu.VMEM((CH, HH), _f32),
        pltpu.VMEM((CH, HH), _f32),
        pltpu.VMEM((CH, HH), _f32),
        pltpu.VMEM((2, HH), _f32),
        pltpu.VMEM_SHARED((N_NODES, HH), _f32),
        pltpu.SemaphoreType.DMA,
        pltpu.SemaphoreType.DMA,
        pltpu.SemaphoreType.DMA,
        pltpu.SemaphoreType.DMA,
    ]
    out_type = [
        jax.ShapeDtypeStruct((N_EDGES, HH), _f32),
        jax.ShapeDtypeStruct((N_EDGES, HH), _f32),
        jax.ShapeDtypeStruct((N_NODES, HH), _f32),
        jax.ShapeDtypeStruct((N_NODES, HH), _f32),
        jax.ShapeDtypeStruct((NS, 2, HH), _f32),
        jax.ShapeDtypeStruct((NS, 2, HH), _f32),
    ]

    @functools.partial(pl.kernel, out_type=out_type, mesh=_mesh(),
                       scratch_types=scratch)
    def edge_a(*refs):
        (src_h, dst_h, et_h, sT0, sT1, dT0, dT1, cT0, cT1) = refs[:9]
        xs = refs[9:9 + n_x]
        zeros_h = refs[9 + n_x]
        (z0_o, z1_o, n0_o, n1_o, st0_o, st1_o) = refs[10 + n_x:16 + n_x]
        (si2, di2, ei2, srows, drows, crows, xrows, z_v, sb_v,
         stats_v, acc_sh, sem_i, sem_g, sem_z, sem_s) = refs[16 + n_x:]
        cid = lax.axis_index("c")
        sid = lax.axis_index("s")

        def run(sT, dT, cT, xT, z_o, n_o, st_o):
            _acc_rows_io(lambda s: zeros_h.at[s], lambda s: acc_sh.at[s], sid)
            plsc.subcore_barrier()
            ebase = sid * EPW

            def fire_idx(base, b):
                pltpu.async_copy(src_h.at[pl.ds(base, CH)], si2.at[b], sem_i)
                pltpu.async_copy(dst_h.at[pl.ds(base, CH)], di2.at[b], sem_i)
                pltpu.async_copy(et_h.at[pl.ds(base, CH)], ei2.at[b], sem_i)

            def drain_idx(base, b):
                pltpu.make_async_copy(src_h.at[pl.ds(base, CH)], si2.at[b], sem_i).wait()
                pltpu.make_async_copy(dst_h.at[pl.ds(base, CH)], di2.at[b], sem_i).wait()
                pltpu.make_async_copy(et_h.at[pl.ds(base, CH)], ei2.at[b], sem_i).wait()

            fire_idx(ebase, 0)

            def chunk_pair(g, carry):
                for b in range(2):
                    i = 2 * g + b
                    base = ebase + i * CH
                    drain_idx(base, b)
                    g1 = pltpu.async_copy(sT.at[si2.at[b]], srows, sem_g)
                    g2 = pltpu.async_copy(dT.at[di2.at[b]], drows, sem_g)
                    g3 = pltpu.async_copy(cT.at[ei2.at[b]], crows, sem_g)
                    if with_x:
                        g4 = pltpu.async_copy(xT.at[pl.ds(base, CH)], xrows, sem_g)

                    @pl.when(i + 1 < NCH)
                    def _():
                        fire_idx(ebase + (i + 1) * CH, 1 - b)

                    g1.wait()
                    g2.wait()
                    g3.wait()
                    if with_x:
                        g4.wait()

                    @pl.when(i >= 1)
                    def _():
                        pltpu.make_async_copy(z_v, z_o.at[pl.ds(base, CH)], sem_z).wait()
                        pltpu.make_async_copy(z_o.at[pl.ds(base, CH)], sb_v, sem_s).wait()

                    def edge(j, car):
                        new = []
                        for r in range(8):
                            sl = pl.ds(r * L, L)
                            z = srows[j, sl] + drows[j, sl] + crows[j, sl]
                            if with_x:
                                z = z + xrows[j, sl]
                            sig = _sigmoid(z)
                            z_v[j, sl] = z
                            sb_v[j, sl] = sig * srows[j, pl.ds(HH + r * L, L)]
                            new.append(car[2 * r] + z)
                            new.append(car[2 * r + 1] + z * z)
                        return tuple(new)

                    carry = lax.fori_loop(0, CH, edge, carry)
                    pltpu.async_copy(z_v, z_o.at[pl.ds(base, CH)], sem_z)
                    pltpu.async_copy(sb_v, acc_sh.at[di2.at[b]], sem_s, add=True)
                return carry

            zero = jnp.zeros((L,), _f32)
            carry = lax.fori_loop(0, NCH // 2, chunk_pair,
                                  tuple(zero for _ in range(16)))
            pltpu.make_async_copy(z_v, z_o.at[pl.ds(ebase, CH)], sem_z).wait()
            pltpu.make_async_copy(z_o.at[pl.ds(ebase, CH)], sb_v, sem_s).wait()
            for r in range(8):
                stats_v[0, pl.ds(r * L, L)] = carry[2 * r]
                stats_v[1, pl.ds(r * L, L)] = carry[2 * r + 1]
            pltpu.sync_copy(stats_v, st_o.at[sid])
            plsc.subcore_barrier()
            _acc_rows_io(lambda s: acc_sh.at[s], lambda s: n_o.at[s], sid)

        x0 = xs[0] if with_x else None
        x1 = xs[1] if with_x else None

        @pl.when(cid == 0)
        def _():
            run(sT0, dT0, cT0, x0, z0_o, n0_o, st0_o)

        @pl.when(cid == 1)
        def _():
            run(sT1, dT1, cT1, x1, z1_o, n1_o, st1_o)

    return edge_a


CHB = 80
NCHB = EPW // CHB


@functools.lru_cache(maxsize=None)
def _edge_pass_b_kernel():
  @functools.partial(
    pl.kernel,
    out_type=[
        jax.ShapeDtypeStruct((N_NODES, HH), _f32),
        jax.ShapeDtypeStruct((N_NODES, HH), _f32),
    ],
    mesh=_mesh(),
    scratch_types=[
        pltpu.VMEM((NCHB, CHB), jnp.int32),
        pltpu.VMEM((2, CHB, HH), _f32),
        pltpu.VMEM_SHARED((N_NODES, HH), _f32),
        pltpu.SemaphoreType.DMA,
        pltpu.SemaphoreType.DMA,
    ],
  )
  def _edge_pass_b(dst3, zT0, zT1, zeros_h, d0_o, d1_o,
                   dst_all, zr2, acc_sh, sem_r, sem_s):
    cid = lax.axis_index("c")
    sid = lax.axis_index("s")

    def run(zT, d_o):
        _acc_rows_io(lambda s: zeros_h.at[s], lambda s: acc_sh.at[s], sid)
        pltpu.sync_copy(dst3.at[sid], dst_all)
        plsc.subcore_barrier()
        ebase = sid * EPW
        pltpu.async_copy(zT.at[pl.ds(ebase, CHB)], zr2.at[0], sem_r)

        def do_chunk(i, b):
            base = ebase + i * CHB
            pltpu.make_async_copy(zT.at[pl.ds(base, CHB)], zr2.at[b], sem_r).wait()

            @pl.when(i >= 1)
            def _():
                pltpu.make_async_copy(zT.at[pl.ds(base, CHB)], zr2.at[b], sem_s).wait()

            @pl.when(i + 1 < NCHB)
            def _():
                pltpu.async_copy(zT.at[pl.ds(base + CHB, CHB)], zr2.at[1 - b], sem_r)

            zr = zr2.at[b]

            def edge(j, __):
                for r in range(8):
                    sl = pl.ds(r * L, L)
                    zr[j, sl] = _sigmoid(zr[j, sl])
                return 0

            lax.fori_loop(0, CHB, edge, 0)
            pltpu.async_copy(zr, acc_sh.at[dst_all.at[i]], sem_s, add=True)

        def chunk_pair(g, _):
            for b in range(2):
                do_chunk(2 * g + b, b)
            return 0

        lax.fori_loop(0, NCHB // 2, chunk_pair, 0)
        if NCHB % 2:
            base = ebase + (NCHB - 1) * CHB
            pltpu.make_async_copy(zT.at[pl.ds(base, CHB)], zr2.at[0], sem_r).wait()
            pltpu.make_async_copy(zT.at[pl.ds(base, CHB)], zr2.at[0], sem_s).wait()
            zr = zr2.at[0]

            def edge_t(j, __):
                for r in range(8):
                    sl = pl.ds(r * L, L)
                    zr[j, sl] = _sigmoid(zr[j, sl])
                return 0

            lax.fori_loop(0, CHB, edge_t, 0)
            pltpu.async_copy(zr, acc_sh.at[dst_all.at[NCHB - 1]], sem_s, add=True)
        pltpu.make_async_copy(zT.at[pl.ds(ebase, CHB)], zr2.at[0], sem_s).wait()
        plsc.subcore_barrier()
        _acc_rows_io(lambda s: acc_sh.at[s], lambda s: d_o.at[s], sid)

    @pl.when(cid == 0)
    def _():
        run(zT0, d0_o)

    @pl.when(cid == 1)
    def _():
        run(zT1, d1_o)

  return _edge_pass_b


CHR = 40
NCHR = EPW // CHR


@functools.lru_cache(maxsize=None)
def _rel_pass_kernel():
  @functools.partial(
    pl.kernel,
    out_type=[
        jax.ShapeDtypeStruct((ETP, HH), _f32),
        jax.ShapeDtypeStruct((ETP, HH), _f32),
        jax.ShapeDtypeStruct((ETP, HH), _f32),
    ],
    mesh=_mesh(),
    scratch_types=[
        pltpu.VMEM((NCHR, CHR), jnp.int32),
        pltpu.VMEM((2, CHR, HH), _f32),
        pltpu.VMEM((2, CHR, HH), _f32),
        pltpu.VMEM((CHR, HH), _f32),
        pltpu.VMEM((CHR, HH), _f32),
        pltpu.VMEM((2, HH), _f32),
        pltpu.VMEM((2, HH), _f32),
        pltpu.VMEM_SHARED((ETP, HH), _f32),
        pltpu.VMEM_SHARED((ETP, HH), _f32),
        pltpu.SemaphoreType.DMA,
        pltpu.SemaphoreType.DMA,
        pltpu.SemaphoreType.DMA,
    ],
  )
  def _rel_pass(et3, z0T0, z0T1, z1T0, z1T1, b00, b01, b10, b11,
                zeros_e, rs0_o, rs1_o, cnt_o,
                et_all, z0r2, z1r2, r_v, ones_v, bn0_v, bn1_v,
                rsum_sh, cnt_sh, sem_r, sem_s, sem_c):
    cid = lax.axis_index("c")
    sid = lax.axis_index("s")

    def run(z0T, z1T, b0, b1, rs_o, count):
        pltpu.sync_copy(zeros_e.at[pl.ds(sid * ERW, ERW)],
                        rsum_sh.at[pl.ds(sid * ERW, ERW)])
        if count:
            pltpu.sync_copy(zeros_e.at[pl.ds(sid * ERW, ERW)],
                            cnt_sh.at[pl.ds(sid * ERW, ERW)])
        pltpu.sync_copy(b0, bn0_v)
        pltpu.sync_copy(b1, bn1_v)
        pltpu.sync_copy(et3.at[sid], et_all)

        if count:
            def fill(j, _):
                for r in range(8):
                    ones_v[j, pl.ds(r * L, L)] = jnp.full((L,), 1.0, _f32)
                return 0

            lax.fori_loop(0, CHR, fill, 0)
        plsc.subcore_barrier()
        ebase = sid * EPW
        pltpu.async_copy(z0T.at[pl.ds(ebase, CHR)], z0r2.at[0], sem_r)
        pltpu.async_copy(z1T.at[pl.ds(ebase, CHR)], z1r2.at[0], sem_r)

        def chunk_pair(g, _):
            for b in range(2):
                i = 2 * g + b
                base = ebase + i * CHR
                pltpu.make_async_copy(z0T.at[pl.ds(base, CHR)], z0r2.at[b], sem_r).wait()
                pltpu.make_async_copy(z1T.at[pl.ds(base, CHR)], z1r2.at[b], sem_r).wait()

                @pl.when(i >= 1)
                def _():
                    pltpu.make_async_copy(z0T.at[pl.ds(base, CHR)], r_v, sem_s).wait()
                    if count:
                        pltpu.make_async_copy(z0T.at[pl.ds(base, CHR)], ones_v, sem_c).wait()

                @pl.when(i + 1 < NCHR)
                def _():
                    pltpu.async_copy(z0T.at[pl.ds(base + CHR, CHR)], z0r2.at[1 - b], sem_r)
                    pltpu.async_copy(z1T.at[pl.ds(base + CHR, CHR)], z1r2.at[1 - b], sem_r)

                z0r = z0r2.at[b]
                z1r = z1r2.at[b]

                def edge(j, __):
                    for r in range(8):
                        sl = pl.ds(r * L, L)
                        a0 = bn0_v[0, sl]
                        c0 = bn0_v[1, sl]
                        a1 = bn1_v[0, sl]
                        c1 = bn1_v[1, sl]
                        r0 = jnp.maximum(a0 * z0r[j, sl] + c0, 0.0)
                        r1 = jnp.maximum(a1 * z1r[j, sl] + c1, 0.0)
                        r_v[j, sl] = r0 + r1
                    return 0

                lax.fori_loop(0, CHR, edge, 0)
                pltpu.async_copy(r_v, rsum_sh.at[et_all.at[i]], sem_s, add=True)
                if count:
                    pltpu.async_copy(ones_v, cnt_sh.at[et_all.at[i]], sem_c, add=True)
            return 0

        lax.fori_loop(0, NCHR // 2, chunk_pair, 0)
        pltpu.make_async_copy(z0T.at[pl.ds(ebase, CHR)], r_v, sem_s).wait()
        if count:
            pltpu.make_async_copy(z0T.at[pl.ds(ebase, CHR)], ones_v, sem_c).wait()
        plsc.subcore_barrier()
        pltpu.sync_copy(rsum_sh.at[pl.ds(sid * ERW, ERW)],
                        rs_o.at[pl.ds(sid * ERW, ERW)])
        if count:
            pltpu.sync_copy(cnt_sh.at[pl.ds(sid * ERW, ERW)],
                            cnt_o.at[pl.ds(sid * ERW, ERW)])

    @pl.when(cid == 0)
    def _():
        run(z0T0, z1T0, b00, b10, rs0_o, True)

    @pl.when(cid == 1)
    def _():
        run(z0T1, z1T1, b01, b11, rs1_o, False)

  return _rel_pass


_MB = 1000


def _dense_body(h_ref, w_ref, b_ref, a_ref, d0_ref, d1_ref, s0_ref, s1_ref):
    y = jnp.dot(h_ref[...], w_ref[...], preferred_element_type=_f32)
    y = y + b_ref[...]
    a_ref[...] = y[:, 0:256]
    d0_ref[...] = y[:, 256:384]
    d1_ref[...] = y[:, 384:512]
    s0_ref[...] = jnp.concatenate([y[:, 512:640], y[:, 768:896]], axis=1)
    s1_ref[...] = jnp.concatenate([y[:, 640:768], y[:, 896:1024]], axis=1)


def _dense(h, wcat, bcat):
    n = h.shape[0]
    grid = (n // _MB,)
    return pl.pallas_call(
        _dense_body,
        grid=grid,
        in_specs=[
            pl.BlockSpec((_MB, HID), lambda i: (i, 0)),
            pl.BlockSpec((HID, 4 * HID), lambda i: (0, 0)),
            pl.BlockSpec((1, 4 * HID), lambda i: (0, 0)),
        ],
        out_specs=[
            pl.BlockSpec((_MB, HID), lambda i: (i, 0)),
            pl.BlockSpec((_MB, HH), lambda i: (i, 0)),
            pl.BlockSpec((_MB, HH), lambda i: (i, 0)),
            pl.BlockSpec((_MB, HID), lambda i: (i, 0)),
            pl.BlockSpec((_MB, HID), lambda i: (i, 0)),
        ],
        out_shape=[
            jax.ShapeDtypeStruct((n, HID), _f32),
            jax.ShapeDtypeStruct((n, HH), _f32),
            jax.ShapeDtypeStruct((n, HH), _f32),
            jax.ShapeDtypeStruct((n, HID), _f32),
            jax.ShapeDtypeStruct((n, HID), _f32),
        ],
    )(h, wcat, bcat)


def _cet_body(e_ref, w0_ref, b0_ref, w1_ref, b1_ref, c00, c01, c10, c11):
    y0 = jnp.dot(e_ref[...], w0_ref[...], preferred_element_type=_f32) + b0_ref[...]
    y1 = jnp.dot(e_ref[...], w1_ref[...], preferred_element_type=_f32) + b1_ref[...]
    c00[...] = y0[:, :HH]
    c01[...] = y0[:, HH:]
    c10[...] = y1[:, :HH]
    c11[...] = y1[:, HH:]


def _cet(emb_e_pad, cw0, cb0, cw1, cb1):
    return pl.pallas_call(
        _cet_body,
        out_shape=[jax.ShapeDtypeStruct((ETP, HH), _f32)] * 4,
    )(emb_e_pad, cw0, cb0.reshape(1, HID), cw1, cb1.reshape(1, HID))


def _node_body(with_stats, *refs):
    if with_stats:
        (h_ref, a_ref, n0, n1, d0, d1, g_ref, b_ref, st0, st1,
         ge_ref, be_ref, out_ref, bn0_ref, bn1_ref) = refs
    else:
        (h_ref, a_ref, n0, n1, d0, d1, g_ref, b_ref, out_ref) = refs
    num = jnp.concatenate([n0[...], n1[...]], axis=1)
    den = jnp.concatenate([d0[...], d1[...]], axis=1)
    x = a_ref[...] + num / (den + 1e-6)
    mu = jnp.mean(x, axis=0, keepdims=True)
    var = jnp.mean((x - mu) ** 2, axis=0, keepdims=True)
    y = g_ref[...] * (x - mu) / jnp.sqrt(var + 1e-5) + b_ref[...]
    out_ref[...] = h_ref[...] + jnp.maximum(y, 0.0)
    if with_stats:
        s0 = jnp.sum(st0[...], axis=0)
        s1 = jnp.sum(st1[...], axis=0)
        for s, ref, lo in ((s0, bn0_ref, 0), (s1, bn1_ref, HH)):
            mean = s[0:1] / N_EDGES
            v = s[1:2] / N_EDGES - mean * mean
            aa = ge_ref[:, lo:lo + HH] / jnp.sqrt(v + 1e-5)
            cc = be_ref[:, lo:lo + HH] - aa * mean
            ref[...] = jnp.concatenate([aa, cc], axis=0)


def _node_update(h_in, a, n0, n1, d0, d1, g, b, stats=None, ge=None, be=None):
    n = h_in.shape[0]
    if stats is None:
        out_shape = jax.ShapeDtypeStruct((n, HID), _f32)
        args = (h_in, a, n0, n1, d0, d1, g.reshape(1, HID), b.reshape(1, HID))
        body = functools.partial(_node_body, False)
    else:
        out_shape = [
            jax.ShapeDtypeStruct((n, HID), _f32),
            jax.ShapeDtypeStruct((2, HH), _f32),
            jax.ShapeDtypeStruct((2, HH), _f32),
        ]
        args = (h_in, a, n0, n1, d0, d1, g.reshape(1, HID), b.reshape(1, HID),
                stats[0], stats[1], ge.reshape(1, HID), be.reshape(1, HID))
        body = functools.partial(_node_body, True)
    return pl.pallas_call(
        body,
        out_shape=out_shape,
        compiler_params=pltpu.CompilerParams(
            vmem_limit_bytes=110 * 1024 * 1024),
    )(*args)


_CMB = 1600


def _cm_body(z0_ref, z1_ref, st0_ref, st1_ref, cw_ref, ge_ref, be_ref,
             x0_ref, x1_ref, bn0_ref, bn1_ref):
    s0 = jnp.sum(st0_ref[...], axis=0)
    s1 = jnp.sum(st1_ref[...], axis=0)
    mean = jnp.concatenate([s0[0:1], s1[0:1]], axis=1) / N_EDGES
    var = jnp.concatenate([s0[1:2], s1[1:2]], axis=1) / N_EDGES - mean * mean
    a = ge_ref[...] / jnp.sqrt(var + 1e-5)
    c = be_ref[...] - a * mean
    z = jnp.concatenate([z0_ref[...], z1_ref[...]], axis=1)
    r = jnp.maximum(a * z + c, 0.0)
    x = jnp.dot(r, cw_ref[...], preferred_element_type=_f32)
    x0_ref[...] = x[:, :HH]
    x1_ref[...] = x[:, HH:]
    bn0_ref[...] = jnp.concatenate([a[:, :HH], c[:, :HH]], axis=0)
    bn1_ref[...] = jnp.concatenate([a[:, HH:], c[:, HH:]], axis=0)


def _cmatmul(z0, z1, st0, st1, cw, ge, be):
    grid = (N_EDGES // _CMB,)
    return pl.pallas_call(
        _cm_body,
        grid=grid,
        in_specs=[
            pl.BlockSpec((_CMB, HH), lambda i: (i, 0)),
            pl.BlockSpec((_CMB, HH), lambda i: (i, 0)),
            pl.BlockSpec((NS, 2, HH), lambda i: (0, 0, 0)),
            pl.BlockSpec((NS, 2, HH), lambda i: (0, 0, 0)),
            pl.BlockSpec((HID, HID), lambda i: (0, 0)),
            pl.BlockSpec((1, HID), lambda i: (0, 0)),
            pl.BlockSpec((1, HID), lambda i: (0, 0)),
        ],
        out_specs=[
            pl.BlockSpec((_CMB, HH), lambda i: (i, 0)),
            pl.BlockSpec((_CMB, HH), lambda i: (i, 0)),
            pl.BlockSpec((2, HH), lambda i: (0, 0)),
            pl.BlockSpec((2, HH), lambda i: (0, 0)),
        ],
        out_shape=[
            jax.ShapeDtypeStruct((N_EDGES, HH), _f32),
            jax.ShapeDtypeStruct((N_EDGES, HH), _f32),
            jax.ShapeDtypeStruct((2, HH), _f32),
            jax.ShapeDtypeStruct((2, HH), _f32),
        ],
    )(z0, z1, st0, st1, cw, ge.reshape(1, HID), be.reshape(1, HID))


def _rel_body(r0_ref, r1_ref, cnt_ref, emb_ref, out_ref):
    cnt = cnt_ref[:, 0:1]
    rsum = jnp.concatenate([r0_ref[...], r1_ref[...]], axis=1)
    total = cnt * emb_ref[...] + rsum
    out_ref[...] = total / jnp.maximum(cnt, 1.0)


def _rel_means(r0, r1, cnt, emb_e_pad):
    return pl.pallas_call(
        _rel_body,
        out_shape=jax.ShapeDtypeStruct((ETP, HID), _f32),
    )(r0, r1, cnt, emb_e_pad)


def kernel(node_id, edge_index, edge_type, p, emb_h, emb_e,
           Aw, Ab, Bw, Bb, Cw, Cb, Dw, Db, Ew, Eb, gh, bh, ge, be):
    src = edge_index[0].astype(jnp.int32)
    dst = edge_index[1].astype(jnp.int32)
    et = edge_type.astype(jnp.int32)

    nid_pad = jnp.concatenate(
        [node_id.astype(jnp.int32), jnp.zeros((NID_PAD - N_NODES,), jnp.int32)])
    emb_e_pad = jnp.concatenate(
        [emb_e, jnp.zeros((ETP - emb_e.shape[0], HID), _f32)], axis=0)

    zeros_n = jnp.zeros((N_NODES, HH), _f32)
    dst3 = dst.reshape(NS, NCHB, CHB)
    et3 = et.reshape(NS, NCHR, CHR)
    zeros_e = jnp.zeros((ETP, HH), _f32)

    h0 = _gather_h0_kernel()(emb_h, nid_pad)[:N_NODES]

    cet = _cet(emb_e_pad, Cw[0], Cb[0], Cw[1], Cb[1])

    def wcat(l):
        w = jnp.concatenate([Aw[l], Ew[l], Dw[l], Bw[l]], axis=1)
        bb = jnp.concatenate([Ab[l], Eb[l], Db[l], Bb[l]]).reshape(1, 4 * HID)
        return w, bb

    w0, b0 = wcat(0)
    a0, e00, e01, s00, s01 = _dense(h0, w0, b0)
    z00, z01, n00, n01, st00, st01 = _make_edge_pass_a(False)(
        src, dst, et, s00, s01, e00, e01, cet[0], cet[1], zeros_n)
    d00, d01 = _edge_pass_b_kernel()(dst3, z00, z01, zeros_n)
    h1 = _node_update(h0, a0, n00, n01, d00, d01, gh[0], bh[0])

    w1, b1 = wcat(1)
    a1, e10, e11, s10, s11 = _dense(h1, w1, b1)
    x0, x1, bn00, bn01 = _cmatmul(z00, z01, st00, st01, Cw[1], ge[0], be[0])
    z10, z11, n10, n11, st10, st11 = _make_edge_pass_a(True)(
        src, dst, et, s10, s11, e10, e11, cet[2], cet[3], x0, x1, zeros_n)
    d10, d11 = _edge_pass_b_kernel()(dst3, z10, z11, zeros_n)
    h2, bn10, bn11 = _node_update(h1, a1, n10, n11, d10, d11, gh[1], bh[1],
                                  stats=(st10, st11), ge=ge[1], be=be[1])

    rs0, rs1, cnt = _rel_pass_kernel()(et3, z00, z01, z10, z11,
                                       bn00, bn01, bn10, bn11, zeros_e)
    rel = _rel_means(rs0, rs1, cnt, emb_e_pad)[:emb_e.shape[0]]
    return (h2, rel)


# speedup vs baseline: 1.2947x; 1.0152x over previous
import functools

import jax
import jax.numpy as jnp
from jax import lax
from jax.experimental import pallas as pl
from jax.experimental.pallas import tpu as pltpu
from jax.experimental.pallas import tpu_sc as plsc

N_NODES = 10000
N_EDGES = 160000
HID = 256
HH = 128
ETP = 512
NC, NS, L = 2, 16, 16
NW = NC * NS

EPW = N_EDGES // NS
CH = 40
NCH = EPW // CH
NPW = N_NODES // NS
ERW = ETP // NS

_f32 = jnp.float32
@functools.lru_cache(maxsize=None)
def _mesh():
    return plsc.VectorSubcoreMesh(core_axis_name="c", subcore_axis_name="s")


def _sigmoid(z):
    return 1.0 / (1.0 + jnp.exp(-z))


_NR8 = 624
_NREM = N_NODES - NS * _NR8


def _acc_rows_io(src_at, dst_at, sid):
    pltpu.sync_copy(src_at(pl.ds(sid * _NR8, _NR8)),
                    dst_at(pl.ds(sid * _NR8, _NR8)))

    @pl.when(sid == NS - 1)
    def _():
        pltpu.sync_copy(src_at(pl.ds(NS * _NR8, _NREM)),
                        dst_at(pl.ds(NS * _NR8, _NREM)))


NID_PAD = 10240
_GPW = NID_PAD // NW


@functools.lru_cache(maxsize=None)
def _gather_h0_kernel():
    @functools.partial(
        pl.kernel,
        out_type=jax.ShapeDtypeStruct((NID_PAD, HID), _f32),
        mesh=_mesh(),
        scratch_types=[
            pltpu.VMEM((_GPW,), jnp.int32),
            pltpu.VMEM((_GPW, HID), _f32),
            pltpu.SemaphoreType.DMA,
        ],
    )
    def _gather_h0(table, nid, out, idx_v, rows_v, sem):
        wid = lax.axis_index("s") * NC + lax.axis_index("c")
        base = wid * _GPW
        pltpu.sync_copy(nid.at[pl.ds(base, _GPW)], idx_v)
        pltpu.async_copy(table.at[idx_v], rows_v, sem).wait()
        pltpu.sync_copy(rows_v, out.at[pl.ds(base, _GPW)])

    return _gather_h0


@functools.lru_cache(maxsize=None)
def _make_edge_pass_a(with_x):
    n_x = 2 if with_x else 0
    scratch = [
        pltpu.VMEM((CH,), jnp.int32),
        pltpu.VMEM((2, CH), jnp.int32),
        pltpu.VMEM((CH,), jnp.int32),
        pltpu.VMEM((2, CH, HID), _f32),
        pltpu.VMEM((2, CH, HH), _f32),
        pltpu.VMEM((CH, HH), _f32),
        pltpu.VMEM((CH, HH), _f32),
        pltpu.VMEM((CH, HH), _f32),
        pltpu.VMEM((2, HH), _f32),
        pltpu.VMEM_SHARED((N_NODES, HH), _f32),
        pltpu.SemaphoreType.DMA,
        pltpu.SemaphoreType.DMA,
        pltpu.SemaphoreType.DMA,
        pltpu.SemaphoreType.DMA,
    ]
    out_type = [
        jax.ShapeDtypeStruct((N_EDGES, HH), _f32),
        jax.ShapeDtypeStruct((N_EDGES, HH), _f32),
        jax.ShapeDtypeStruct((N_NODES, HH), _f32),
        jax.ShapeDtypeStruct((N_NODES, HH), _f32),
        jax.ShapeDtypeStruct((NS, 2, HH), _f32),
        jax.ShapeDtypeStruct((NS, 2, HH), _f32),
    ]

    @functools.partial(pl.kernel, out_type=out_type, mesh=_mesh(),
                       scratch_types=scratch)
    def edge_a(*refs):
        (src_h, dst_h, et_h, sT0, sT1, dT0, dT1, cT0, cT1) = refs[:9]
        xs = refs[9:9 + n_x]
        zeros_h = refs[9 + n_x]
        (z0_o, z1_o, n0_o, n1_o, st0_o, st1_o) = refs[10 + n_x:16 + n_x]
        (si, di2, ei, srows2, drows2, crows, xrows, z_v,
         stats_v, acc_sh, sem_i, sem_g, sem_z, sem_s) = refs[16 + n_x:]
        cid = lax.axis_index("c")
        sid = lax.axis_index("s")

        def run(sT, dT, cT, xT, z_o, n_o, st_o):
            _acc_rows_io(lambda s: zeros_h.at[s], lambda s: acc_sh.at[s], sid)
            plsc.subcore_barrier()
            ebase = sid * EPW

            def fire_idx(base, b):
                pltpu.async_copy(src_h.at[pl.ds(base, CH)], si, sem_i)
                pltpu.async_copy(dst_h.at[pl.ds(base, CH)], di2.at[b], sem_i)
                pltpu.async_copy(et_h.at[pl.ds(base, CH)], ei, sem_i)

            def drain_idx(base, b):
                pltpu.make_async_copy(src_h.at[pl.ds(base, CH)], si, sem_i).wait()
                pltpu.make_async_copy(dst_h.at[pl.ds(base, CH)], di2.at[b], sem_i).wait()
                pltpu.make_async_copy(et_h.at[pl.ds(base, CH)], ei, sem_i).wait()

            def fire_gathers(base, b):
                pltpu.async_copy(sT.at[si], srows2.at[b], sem_g)
                pltpu.async_copy(dT.at[di2.at[b]], drows2.at[b], sem_g)
                pltpu.async_copy(cT.at[ei], crows, sem_g)
                if with_x:
                    pltpu.async_copy(xT.at[pl.ds(base, CH)], xrows, sem_g)

            def drain_gathers(base, b):
                pltpu.make_async_copy(sT.at[si], srows2.at[b], sem_g).wait()
                pltpu.make_async_copy(dT.at[di2.at[b]], drows2.at[b], sem_g).wait()
                pltpu.make_async_copy(cT.at[ei], crows, sem_g).wait()
                if with_x:
                    pltpu.make_async_copy(xT.at[pl.ds(base, CH)], xrows, sem_g).wait()

            fire_idx(ebase, 0)
            drain_idx(ebase, 0)
            fire_gathers(ebase, 0)

            def chunk_pair(g, carry):
                for b in range(2):
                    i = 2 * g + b
                    base = ebase + i * CH
                    drain_gathers(base, b)

                    @pl.when(i >= 1)
                    def _():
                        pltpu.make_async_copy(z_v, z_o.at[pl.ds(base, CH)], sem_z).wait()
                        pltpu.make_async_copy(z_o.at[pl.ds(base, CH)], drows2.at[b], sem_s).wait()

                    @pl.when(i + 1 < NCH)
                    def _():
                        fire_idx(ebase + (i + 1) * CH, 1 - b)

                    sr = srows2.at[b]
                    dr = drows2.at[b]

                    def edge(j, car):
                        new = []
                        for r in range(8):
                            sl = pl.ds(r * L, L)
                            z = sr[j, sl] + dr[j, sl] + crows[j, sl]
                            if with_x:
                                z = z + xrows[j, sl]
                            sig = _sigmoid(z)
                            z_v[j, sl] = z
                            dr[j, sl] = sig * sr[j, pl.ds(HH + r * L, L)]
                            new.append(car[2 * r] + z)
                            new.append(car[2 * r + 1] + z * z)
                        return tuple(new)

                    carry = lax.fori_loop(0, CH, edge, carry)
                    pltpu.async_copy(z_v, z_o.at[pl.ds(base, CH)], sem_z)
                    pltpu.async_copy(dr, acc_sh.at[di2.at[b]], sem_s, add=True)

                    @pl.when(i + 1 < NCH)
                    def _():
                        nbase = ebase + (i + 1) * CH
                        drain_idx(nbase, 1 - b)
                        fire_gathers(nbase, 1 - b)
                return carry

            zero = jnp.zeros((L,), _f32)
            carry = lax.fori_loop(0, NCH // 2, chunk_pair,
                                  tuple(zero for _ in range(16)))
            pltpu.make_async_copy(z_v, z_o.at[pl.ds(ebase, CH)], sem_z).wait()
            pltpu.make_async_copy(z_o.at[pl.ds(ebase, CH)], drows2.at[0], sem_s).wait()
            for r in range(8):
                stats_v[0, pl.ds(r * L, L)] = carry[2 * r]
                stats_v[1, pl.ds(r * L, L)] = carry[2 * r + 1]
            pltpu.sync_copy(stats_v, st_o.at[sid])
            plsc.subcore_barrier()
            _acc_rows_io(lambda s: acc_sh.at[s], lambda s: n_o.at[s], sid)

        x0 = xs[0] if with_x else None
        x1 = xs[1] if with_x else None

        @pl.when(cid == 0)
        def _():
            run(sT0, dT0, cT0, x0, z0_o, n0_o, st0_o)

        @pl.when(cid == 1)
        def _():
            run(sT1, dT1, cT1, x1, z1_o, n1_o, st1_o)

    return edge_a


CHB = 80
NCHB = EPW // CHB


@functools.lru_cache(maxsize=None)
def _edge_pass_b_kernel():
  @functools.partial(
    pl.kernel,
    out_type=[
        jax.ShapeDtypeStruct((N_NODES, HH), _f32),
        jax.ShapeDtypeStruct((N_NODES, HH), _f32),
    ],
    mesh=_mesh(),
    scratch_types=[
        pltpu.VMEM((NCHB, CHB), jnp.int32),
        pltpu.VMEM((2, CHB, HH), _f32),
        pltpu.VMEM_SHARED((N_NODES, HH), _f32),
        pltpu.SemaphoreType.DMA,
        pltpu.SemaphoreType.DMA,
    ],
  )
  def _edge_pass_b(dst3, zT0, zT1, zeros_h, d0_o, d1_o,
                   dst_all, zr2, acc_sh, sem_r, sem_s):
    cid = lax.axis_index("c")
    sid = lax.axis_index("s")

    def run(zT, d_o):
        _acc_rows_io(lambda s: zeros_h.at[s], lambda s: acc_sh.at[s], sid)
        pltpu.sync_copy(dst3.at[sid], dst_all)
        plsc.subcore_barrier()
        ebase = sid * EPW
        pltpu.async_copy(zT.at[pl.ds(ebase, CHB)], zr2.at[0], sem_r)

        def do_chunk(i, b):
            base = ebase + i * CHB
            pltpu.make_async_copy(zT.at[pl.ds(base, CHB)], zr2.at[b], sem_r).wait()

            @pl.when(i >= 1)
            def _():
                pltpu.make_async_copy(zT.at[pl.ds(base, CHB)], zr2.at[b], sem_s).wait()

            @pl.when(i + 1 < NCHB)
            def _():
                pltpu.async_copy(zT.at[pl.ds(base + CHB, CHB)], zr2.at[1 - b], sem_r)

            zr = zr2.at[b]

            def edge(j, __):
                for r in range(8):
                    sl = pl.ds(r * L, L)
                    zr[j, sl] = _sigmoid(zr[j, sl])
                return 0

            lax.fori_loop(0, CHB, edge, 0)
            pltpu.async_copy(zr, acc_sh.at[dst_all.at[i]], sem_s, add=True)

        def chunk_pair(g, _):
            for b in range(2):
                do_chunk(2 * g + b, b)
            return 0

        lax.fori_loop(0, NCHB // 2, chunk_pair, 0)
        if NCHB % 2:
            base = ebase + (NCHB - 1) * CHB
            pltpu.make_async_copy(zT.at[pl.ds(base, CHB)], zr2.at[0], sem_r).wait()
            pltpu.make_async_copy(zT.at[pl.ds(base, CHB)], zr2.at[0], sem_s).wait()
            zr = zr2.at[0]

            def edge_t(j, __):
                for r in range(8):
                    sl = pl.ds(r * L, L)
                    zr[j, sl] = _sigmoid(zr[j, sl])
                return 0

            lax.fori_loop(0, CHB, edge_t, 0)
            pltpu.async_copy(zr, acc_sh.at[dst_all.at[NCHB - 1]], sem_s, add=True)
        pltpu.make_async_copy(zT.at[pl.ds(ebase, CHB)], zr2.at[0], sem_s).wait()
        plsc.subcore_barrier()
        _acc_rows_io(lambda s: acc_sh.at[s], lambda s: d_o.at[s], sid)

    @pl.when(cid == 0)
    def _():
        run(zT0, d0_o)

    @pl.when(cid == 1)
    def _():
        run(zT1, d1_o)

  return _edge_pass_b


CHR = 40
NCHR = EPW // CHR


@functools.lru_cache(maxsize=None)
def _rel_pass_kernel():
  @functools.partial(
    pl.kernel,
    out_type=[
        jax.ShapeDtypeStruct((ETP, HH), _f32),
        jax.ShapeDtypeStruct((ETP, HH), _f32),
        jax.ShapeDtypeStruct((ETP, HH), _f32),
    ],
    mesh=_mesh(),
    scratch_types=[
        pltpu.VMEM((NCHR, CHR), jnp.int32),
        pltpu.VMEM((2, CHR, HH), _f32),
        pltpu.VMEM((2, CHR, HH), _f32),
        pltpu.VMEM((CHR, HH), _f32),
        pltpu.VMEM((CHR, HH), _f32),
        pltpu.VMEM((2, HH), _f32),
        pltpu.VMEM((2, HH), _f32),
        pltpu.VMEM_SHARED((ETP, HH), _f32),
        pltpu.VMEM_SHARED((ETP, HH), _f32),
        pltpu.SemaphoreType.DMA,
        pltpu.SemaphoreType.DMA,
        pltpu.SemaphoreType.DMA,
    ],
  )
  def _rel_pass(et3, z0T0, z0T1, z1T0, z1T1, b00, b01, b10, b11,
                zeros_e, rs0_o, rs1_o, cnt_o,
                et_all, z0r2, z1r2, r_v, ones_v, bn0_v, bn1_v,
                rsum_sh, cnt_sh, sem_r, sem_s, sem_c):
    cid = lax.axis_index("c")
    sid = lax.axis_index("s")

    def run(z0T, z1T, b0, b1, rs_o, count):
        pltpu.sync_copy(zeros_e.at[pl.ds(sid * ERW, ERW)],
                        rsum_sh.at[pl.ds(sid * ERW, ERW)])
        if count:
            pltpu.sync_copy(zeros_e.at[pl.ds(sid * ERW, ERW)],
                            cnt_sh.at[pl.ds(sid * ERW, ERW)])
        pltpu.sync_copy(b0, bn0_v)
        pltpu.sync_copy(b1, bn1_v)
        pltpu.sync_copy(et3.at[sid], et_all)

        if count:
            def fill(j, _):
                for r in range(8):
                    ones_v[j, pl.ds(r * L, L)] = jnp.full((L,), 1.0, _f32)
                return 0

            lax.fori_loop(0, CHR, fill, 0)
        plsc.subcore_barrier()
        ebase = sid * EPW
        pltpu.async_copy(z0T.at[pl.ds(ebase, CHR)], z0r2.at[0], sem_r)
        pltpu.async_copy(z1T.at[pl.ds(ebase, CHR)], z1r2.at[0], sem_r)

        def chunk_pair(g, _):
            for b in range(2):
                i = 2 * g + b
                base = ebase + i * CHR
                pltpu.make_async_copy(z0T.at[pl.ds(base, CHR)], z0r2.at[b], sem_r).wait()
                pltpu.make_async_copy(z1T.at[pl.ds(base, CHR)], z1r2.at[b], sem_r).wait()

                @pl.when(i >= 1)
                def _():
                    pltpu.make_async_copy(z0T.at[pl.ds(base, CHR)], r_v, sem_s).wait()
                    if count:
                        pltpu.make_async_copy(z0T.at[pl.ds(base, CHR)], ones_v, sem_c).wait()

                @pl.when(i + 1 < NCHR)
                def _():
                    pltpu.async_copy(z0T.at[pl.ds(base + CHR, CHR)], z0r2.at[1 - b], sem_r)
                    pltpu.async_copy(z1T.at[pl.ds(base + CHR, CHR)], z1r2.at[1 - b], sem_r)

                z0r = z0r2.at[b]
                z1r = z1r2.at[b]

                def edge(j, __):
                    for r in range(8):
                        sl = pl.ds(r * L, L)
                        a0 = bn0_v[0, sl]
                        c0 = bn0_v[1, sl]
                        a1 = bn1_v[0, sl]
                        c1 = bn1_v[1, sl]
                        r0 = jnp.maximum(a0 * z0r[j, sl] + c0, 0.0)
                        r1 = jnp.maximum(a1 * z1r[j, sl] + c1, 0.0)
                        r_v[j, sl] = r0 + r1
                    return 0

                lax.fori_loop(0, CHR, edge, 0)
                pltpu.async_copy(r_v, rsum_sh.at[et_all.at[i]], sem_s, add=True)
                if count:
                    pltpu.async_copy(ones_v, cnt_sh.at[et_all.at[i]], sem_c, add=True)
            return 0

        lax.fori_loop(0, NCHR // 2, chunk_pair, 0)
        pltpu.make_async_copy(z0T.at[pl.ds(ebase, CHR)], r_v, sem_s).wait()
        if count:
            pltpu.make_async_copy(z0T.at[pl.ds(ebase, CHR)], ones_v, sem_c).wait()
        plsc.subcore_barrier()
        pltpu.sync_copy(rsum_sh.at[pl.ds(sid * ERW, ERW)],
                        rs_o.at[pl.ds(sid * ERW, ERW)])
        if count:
            pltpu.sync_copy(cnt_sh.at[pl.ds(sid * ERW, ERW)],
                            cnt_o.at[pl.ds(sid * ERW, ERW)])

    @pl.when(cid == 0)
    def _():
        run(z0T0, z1T0, b00, b10, rs0_o, True)

    @pl.when(cid == 1)
    def _():
        run(z0T1, z1T1, b01, b11, rs1_o, False)

  return _rel_pass


_MB = 1000


def _dense_body(h_ref, w_ref, b_ref, a_ref, d0_ref, d1_ref, s0_ref, s1_ref):
    y = jnp.dot(h_ref[...], w_ref[...], preferred_element_type=_f32)
    y = y + b_ref[...]
    a_ref[...] = y[:, 0:256]
    d0_ref[...] = y[:, 256:384]
    d1_ref[...] = y[:, 384:512]
    s0_ref[...] = jnp.concatenate([y[:, 512:640], y[:, 768:896]], axis=1)
    s1_ref[...] = jnp.concatenate([y[:, 640:768], y[:, 896:1024]], axis=1)


def _dense(h, wcat, bcat):
    n = h.shape[0]
    grid = (n // _MB,)
    return pl.pallas_call(
        _dense_body,
        grid=grid,
        in_specs=[
            pl.BlockSpec((_MB, HID), lambda i: (i, 0)),
            pl.BlockSpec((HID, 4 * HID), lambda i: (0, 0)),
            pl.BlockSpec((1, 4 * HID), lambda i: (0, 0)),
        ],
        out_specs=[
            pl.BlockSpec((_MB, HID), lambda i: (i, 0)),
            pl.BlockSpec((_MB, HH), lambda i: (i, 0)),
            pl.BlockSpec((_MB, HH), lambda i: (i, 0)),
            pl.BlockSpec((_MB, HID), lambda i: (i, 0)),
            pl.BlockSpec((_MB, HID), lambda i: (i, 0)),
        ],
        out_shape=[
            jax.ShapeDtypeStruct((n, HID), _f32),
            jax.ShapeDtypeStruct((n, HH), _f32),
            jax.ShapeDtypeStruct((n, HH), _f32),
            jax.ShapeDtypeStruct((n, HID), _f32),
            jax.ShapeDtypeStruct((n, HID), _f32),
        ],
    )(h, wcat, bcat)


def _cet_body(e_ref, w0_ref, b0_ref, w1_ref, b1_ref, c00, c01, c10, c11):
    y0 = jnp.dot(e_ref[...], w0_ref[...], preferred_element_type=_f32) + b0_ref[...]
    y1 = jnp.dot(e_ref[...], w1_ref[...], preferred_element_type=_f32) + b1_ref[...]
    c00[...] = y0[:, :HH]
    c01[...] = y0[:, HH:]
    c10[...] = y1[:, :HH]
    c11[...] = y1[:, HH:]


def _cet(emb_e_pad, cw0, cb0, cw1, cb1):
    return pl.pallas_call(
        _cet_body,
        out_shape=[jax.ShapeDtypeStruct((ETP, HH), _f32)] * 4,
    )(emb_e_pad, cw0, cb0.reshape(1, HID), cw1, cb1.reshape(1, HID))


def _node_body(with_stats, *refs):
    if with_stats:
        (h_ref, a_ref, n0, n1, d0, d1, g_ref, b_ref, st0, st1,
         ge_ref, be_ref, out_ref, bn0_ref, bn1_ref) = refs
    else:
        (h_ref, a_ref, n0, n1, d0, d1, g_ref, b_ref, out_ref) = refs
    num = jnp.concatenate([n0[...], n1[...]], axis=1)
    den = jnp.concatenate([d0[...], d1[...]], axis=1)
    x = a_ref[...] + num / (den + 1e-6)
    mu = jnp.mean(x, axis=0, keepdims=True)
    var = jnp.mean((x - mu) ** 2, axis=0, keepdims=True)
    y = g_ref[...] * (x - mu) / jnp.sqrt(var + 1e-5) + b_ref[...]
    out_ref[...] = h_ref[...] + jnp.maximum(y, 0.0)
    if with_stats:
        s0 = jnp.sum(st0[...], axis=0)
        s1 = jnp.sum(st1[...], axis=0)
        for s, ref, lo in ((s0, bn0_ref, 0), (s1, bn1_ref, HH)):
            mean = s[0:1] / N_EDGES
            v = s[1:2] / N_EDGES - mean * mean
            aa = ge_ref[:, lo:lo + HH] / jnp.sqrt(v + 1e-5)
            cc = be_ref[:, lo:lo + HH] - aa * mean
            ref[...] = jnp.concatenate([aa, cc], axis=0)


def _node_update(h_in, a, n0, n1, d0, d1, g, b, stats=None, ge=None, be=None):
    n = h_in.shape[0]
    if stats is None:
        out_shape = jax.ShapeDtypeStruct((n, HID), _f32)
        args = (h_in, a, n0, n1, d0, d1, g.reshape(1, HID), b.reshape(1, HID))
        body = functools.partial(_node_body, False)
    else:
        out_shape = [
            jax.ShapeDtypeStruct((n, HID), _f32),
            jax.ShapeDtypeStruct((2, HH), _f32),
            jax.ShapeDtypeStruct((2, HH), _f32),
        ]
        args = (h_in, a, n0, n1, d0, d1, g.reshape(1, HID), b.reshape(1, HID),
                stats[0], stats[1], ge.reshape(1, HID), be.reshape(1, HID))
        body = functools.partial(_node_body, True)
    return pl.pallas_call(
        body,
        out_shape=out_shape,
        compiler_params=pltpu.CompilerParams(
            vmem_limit_bytes=110 * 1024 * 1024),
    )(*args)


_CMB = 1600


def _cm_body(z0_ref, z1_ref, st0_ref, st1_ref, cw_ref, ge_ref, be_ref,
             x0_ref, x1_ref, bn0_ref, bn1_ref):
    s0 = jnp.sum(st0_ref[...], axis=0)
    s1 = jnp.sum(st1_ref[...], axis=0)
    mean = jnp.concatenate([s0[0:1], s1[0:1]], axis=1) / N_EDGES
    var = jnp.concatenate([s0[1:2], s1[1:2]], axis=1) / N_EDGES - mean * mean
    a = ge_ref[...] / jnp.sqrt(var + 1e-5)
    c = be_ref[...] - a * mean
    z = jnp.concatenate([z0_ref[...], z1_ref[...]], axis=1)
    r = jnp.maximum(a * z + c, 0.0)
    x = jnp.dot(r, cw_ref[...], preferred_element_type=_f32)
    x0_ref[...] = x[:, :HH]
    x1_ref[...] = x[:, HH:]
    bn0_ref[...] = jnp.concatenate([a[:, :HH], c[:, :HH]], axis=0)
    bn1_ref[...] = jnp.concatenate([a[:, HH:], c[:, HH:]], axis=0)


def _cmatmul(z0, z1, st0, st1, cw, ge, be):
    grid = (N_EDGES // _CMB,)
    return pl.pallas_call(
        _cm_body,
        grid=grid,
        in_specs=[
            pl.BlockSpec((_CMB, HH), lambda i: (i, 0)),
            pl.BlockSpec((_CMB, HH), lambda i: (i, 0)),
            pl.BlockSpec((NS, 2, HH), lambda i: (0, 0, 0)),
            pl.BlockSpec((NS, 2, HH), lambda i: (0, 0, 0)),
            pl.BlockSpec((HID, HID), lambda i: (0, 0)),
            pl.BlockSpec((1, HID), lambda i: (0, 0)),
            pl.BlockSpec((1, HID), lambda i: (0, 0)),
        ],
        out_specs=[
            pl.BlockSpec((_CMB, HH), lambda i: (i, 0)),
            pl.BlockSpec((_CMB, HH), lambda i: (i, 0)),
            pl.BlockSpec((2, HH), lambda i: (0, 0)),
            pl.BlockSpec((2, HH), lambda i: (0, 0)),
        ],
        out_shape=[
            jax.ShapeDtypeStruct((N_EDGES, HH), _f32),
            jax.ShapeDtypeStruct((N_EDGES, HH), _f32),
            jax.ShapeDtypeStruct((2, HH), _f32),
            jax.ShapeDtypeStruct((2, HH), _f32),
        ],
    )(z0, z1, st0, st1, cw, ge.reshape(1, HID), be.reshape(1, HID))


def _rel_body(r0_ref, r1_ref, cnt_ref, emb_ref, out_ref):
    cnt = cnt_ref[:, 0:1]
    rsum = jnp.concatenate([r0_ref[...], r1_ref[...]], axis=1)
    total = cnt * emb_ref[...] + rsum
    out_ref[...] = total / jnp.maximum(cnt, 1.0)


def _rel_means(r0, r1, cnt, emb_e_pad):
    return pl.pallas_call(
        _rel_body,
        out_shape=jax.ShapeDtypeStruct((ETP, HID), _f32),
    )(r0, r1, cnt, emb_e_pad)


def kernel(node_id, edge_index, edge_type, p, emb_h, emb_e,
           Aw, Ab, Bw, Bb, Cw, Cb, Dw, Db, Ew, Eb, gh, bh, ge, be):
    src = edge_index[0].astype(jnp.int32)
    dst = edge_index[1].astype(jnp.int32)
    et = edge_type.astype(jnp.int32)

    nid_pad = jnp.concatenate(
        [node_id.astype(jnp.int32), jnp.zeros((NID_PAD - N_NODES,), jnp.int32)])
    emb_e_pad = jnp.concatenate(
        [emb_e, jnp.zeros((ETP - emb_e.shape[0], HID), _f32)], axis=0)

    zeros_n = jnp.zeros((N_NODES, HH), _f32)
    dst3 = dst.reshape(NS, NCHB, CHB)
    et3 = et.reshape(NS, NCHR, CHR)
    zeros_e = jnp.zeros((ETP, HH), _f32)

    h0 = _gather_h0_kernel()(emb_h, nid_pad)[:N_NODES]

    cet = _cet(emb_e_pad, Cw[0], Cb[0], Cw[1], Cb[1])

    def wcat(l):
        w = jnp.concatenate([Aw[l], Ew[l], Dw[l], Bw[l]], axis=1)
        bb = jnp.concatenate([Ab[l], Eb[l], Db[l], Bb[l]]).reshape(1, 4 * HID)
        return w, bb

    w0, b0 = wcat(0)
    a0, e00, e01, s00, s01 = _dense(h0, w0, b0)
    z00, z01, n00, n01, st00, st01 = _make_edge_pass_a(False)(
        src, dst, et, s00, s01, e00, e01, cet[0], cet[1], zeros_n)
    d00, d01 = _edge_pass_b_kernel()(dst3, z00, z01, zeros_n)
    h1 = _node_update(h0, a0, n00, n01, d00, d01, gh[0], bh[0])

    w1, b1 = wcat(1)
    a1, e10, e11, s10, s11 = _dense(h1, w1, b1)
    x0, x1, bn00, bn01 = _cmatmul(z00, z01, st00, st01, Cw[1], ge[0], be[0])
    z10, z11, n10, n11, st10, st11 = _make_edge_pass_a(True)(
        src, dst, et, s10, s11, e10, e11, cet[2], cet[3], x0, x1, zeros_n)
    d10, d11 = _edge_pass_b_kernel()(dst3, z10, z11, zeros_n)
    h2, bn10, bn11 = _node_update(h1, a1, n10, n11, d10, d11, gh[1], bh[1],
                                  stats=(st10, st11), ge=ge[1], be=be[1])

    rs0, rs1, cnt = _rel_pass_kernel()(et3, z00, z01, z10, z11,
                                       bn00, bn01, bn10, bn11, zeros_e)
    rel = _rel_means(rs0, rs1, cnt, emb_e_pad)[:emb_e.shape[0]]
    return (h2, rel)


# speedup vs baseline: 1.2952x; 1.0004x over previous
import functools

import jax
import jax.numpy as jnp
from jax import lax
from jax.experimental import pallas as pl
from jax.experimental.pallas import tpu as pltpu
from jax.experimental.pallas import tpu_sc as plsc

N_NODES = 10000
N_EDGES = 160000
HID = 256
HH = 128
ETP = 512
NC, NS, L = 2, 16, 16
NW = NC * NS

EPW = N_EDGES // NS
CH = 40
NCH = EPW // CH
NPW = N_NODES // NS
ERW = ETP // NS

_f32 = jnp.float32
@functools.lru_cache(maxsize=None)
def _mesh():
    return plsc.VectorSubcoreMesh(core_axis_name="c", subcore_axis_name="s")


def _sigmoid(z):
    return 1.0 / (1.0 + jnp.exp(-z))


_NR8 = 624
_NREM = N_NODES - NS * _NR8


def _acc_rows_io(src_at, dst_at, sid):
    pltpu.sync_copy(src_at(pl.ds(sid * _NR8, _NR8)),
                    dst_at(pl.ds(sid * _NR8, _NR8)))

    @pl.when(sid == NS - 1)
    def _():
        pltpu.sync_copy(src_at(pl.ds(NS * _NR8, _NREM)),
                        dst_at(pl.ds(NS * _NR8, _NREM)))


NID_PAD = 10240
_GPW = NID_PAD // NW


@functools.lru_cache(maxsize=None)
def _gather_h0_kernel():
    @functools.partial(
        pl.kernel,
        out_type=jax.ShapeDtypeStruct((NID_PAD, HID), _f32),
        mesh=_mesh(),
        scratch_types=[
            pltpu.VMEM((_GPW,), jnp.int32),
            pltpu.VMEM((_GPW, HID), _f32),
            pltpu.SemaphoreType.DMA,
        ],
    )
    def _gather_h0(table, nid, out, idx_v, rows_v, sem):
        wid = lax.axis_index("s") * NC + lax.axis_index("c")
        base = wid * _GPW
        pltpu.sync_copy(nid.at[pl.ds(base, _GPW)], idx_v)
        pltpu.async_copy(table.at[idx_v], rows_v, sem).wait()
        pltpu.sync_copy(rows_v, out.at[pl.ds(base, _GPW)])

    return _gather_h0


@functools.lru_cache(maxsize=None)
def _make_edge_pass_a(with_x):
    n_x = 2 if with_x else 0
    scratch = [
        pltpu.VMEM((CH,), jnp.int32),
        pltpu.VMEM((2, CH), jnp.int32),
        pltpu.VMEM((CH,), jnp.int32),
        pltpu.VMEM((2, CH, HID), _f32),
        pltpu.VMEM((2, CH, HH), _f32),
        pltpu.VMEM((CH, HH), _f32),
        pltpu.VMEM((CH, HH), _f32),
        pltpu.VMEM((CH, HH), _f32),
        pltpu.VMEM((2, HH), _f32),
        pltpu.VMEM_SHARED((N_NODES, HH), _f32),
        pltpu.SemaphoreType.DMA,
        pltpu.SemaphoreType.DMA,
        pltpu.SemaphoreType.DMA,
        pltpu.SemaphoreType.DMA,
    ]
    out_type = [
        jax.ShapeDtypeStruct((N_EDGES, HH), _f32),
        jax.ShapeDtypeStruct((N_EDGES, HH), _f32),
        jax.ShapeDtypeStruct((N_NODES, HH), _f32),
        jax.ShapeDtypeStruct((N_NODES, HH), _f32),
        jax.ShapeDtypeStruct((NS, 2, HH), _f32),
        jax.ShapeDtypeStruct((NS, 2, HH), _f32),
    ]

    @functools.partial(pl.kernel, out_type=out_type, mesh=_mesh(),
                       scratch_types=scratch)
    def edge_a(*refs):
        (src_h, dst_h, et_h, sT0, sT1, dT0, dT1, cT0, cT1) = refs[:9]
        xs = refs[9:9 + n_x]
        zeros_h = refs[9 + n_x]
        (z0_o, z1_o, n0_o, n1_o, st0_o, st1_o) = refs[10 + n_x:16 + n_x]
        (si, di2, ei, srows2, drows2, crows, xrows, z_v,
         stats_v, acc_sh, sem_i, sem_g, sem_z, sem_s) = refs[16 + n_x:]
        cid = lax.axis_index("c")
        sid = lax.axis_index("s")

        def run(sT, dT, cT, xT, z_o, n_o, st_o):
            _acc_rows_io(lambda s: zeros_h.at[s], lambda s: acc_sh.at[s], sid)
            plsc.subcore_barrier()
            ebase = sid * EPW

            def fire_idx(base, b):
                pltpu.async_copy(src_h.at[pl.ds(base, CH)], si, sem_i)
                pltpu.async_copy(dst_h.at[pl.ds(base, CH)], di2.at[b], sem_i)
                pltpu.async_copy(et_h.at[pl.ds(base, CH)], ei, sem_i)

            def drain_idx(base, b):
                pltpu.make_async_copy(src_h.at[pl.ds(base, CH)], si, sem_i).wait()
                pltpu.make_async_copy(dst_h.at[pl.ds(base, CH)], di2.at[b], sem_i).wait()
                pltpu.make_async_copy(et_h.at[pl.ds(base, CH)], ei, sem_i).wait()

            def fire_gathers(base, b):
                pltpu.async_copy(sT.at[si], srows2.at[b], sem_g)
                pltpu.async_copy(dT.at[di2.at[b]], drows2.at[b], sem_g)
                pltpu.async_copy(cT.at[ei], crows, sem_g)
                if with_x:
                    pltpu.async_copy(xT.at[pl.ds(base, CH)], xrows, sem_g)

            def drain_gathers(base, b):
                pltpu.make_async_copy(sT.at[si], srows2.at[b], sem_g).wait()
                pltpu.make_async_copy(dT.at[di2.at[b]], drows2.at[b], sem_g).wait()
                pltpu.make_async_copy(cT.at[ei], crows, sem_g).wait()
                if with_x:
                    pltpu.make_async_copy(xT.at[pl.ds(base, CH)], xrows, sem_g).wait()

            fire_idx(ebase, 0)
            drain_idx(ebase, 0)
            fire_gathers(ebase, 0)

            def chunk_pair(g, carry):
                for b in range(2):
                    i = 2 * g + b
                    base = ebase + i * CH
                    drain_gathers(base, b)

                    @pl.when(i >= 1)
                    def _():
                        pltpu.make_async_copy(z_v, z_o.at[pl.ds(base, CH)], sem_z).wait()
                        pltpu.make_async_copy(z_o.at[pl.ds(base, CH)], drows2.at[b], sem_s).wait()

                    @pl.when(i + 1 < NCH)
                    def _():
                        fire_idx(ebase + (i + 1) * CH, 1 - b)

                    sr = srows2.at[b]
                    dr = drows2.at[b]

                    def edge(jp, car):
                        acc = list(car)
                        for jj in range(2):
                            j = jp * 2 + jj
                            for r in range(8):
                                sl = pl.ds(r * L, L)
                                z = sr[j, sl] + dr[j, sl] + crows[j, sl]
                                if with_x:
                                    z = z + xrows[j, sl]
                                sig = _sigmoid(z)
                                z_v[j, sl] = z
                                dr[j, sl] = sig * sr[j, pl.ds(HH + r * L, L)]
                                acc[2 * r] = acc[2 * r] + z
                                acc[2 * r + 1] = acc[2 * r + 1] + z * z
                        return tuple(acc)

                    carry = lax.fori_loop(0, CH // 2, edge, carry)
                    pltpu.async_copy(z_v, z_o.at[pl.ds(base, CH)], sem_z)
                    pltpu.async_copy(dr, acc_sh.at[di2.at[b]], sem_s, add=True)

                    @pl.when(i + 1 < NCH)
                    def _():
                        nbase = ebase + (i + 1) * CH
                        drain_idx(nbase, 1 - b)
                        fire_gathers(nbase, 1 - b)
                return carry

            zero = jnp.zeros((L,), _f32)
            carry = lax.fori_loop(0, NCH // 2, chunk_pair,
                                  tuple(zero for _ in range(16)))
            pltpu.make_async_copy(z_v, z_o.at[pl.ds(ebase, CH)], sem_z).wait()
            pltpu.make_async_copy(z_o.at[pl.ds(ebase, CH)], drows2.at[0], sem_s).wait()
            for r in range(8):
                stats_v[0, pl.ds(r * L, L)] = carry[2 * r]
                stats_v[1, pl.ds(r * L, L)] = carry[2 * r + 1]
            pltpu.sync_copy(stats_v, st_o.at[sid])
            plsc.subcore_barrier()
            _acc_rows_io(lambda s: acc_sh.at[s], lambda s: n_o.at[s], sid)

        x0 = xs[0] if with_x else None
        x1 = xs[1] if with_x else None

        @pl.when(cid == 0)
        def _():
            run(sT0, dT0, cT0, x0, z0_o, n0_o, st0_o)

        @pl.when(cid == 1)
        def _():
            run(sT1, dT1, cT1, x1, z1_o, n1_o, st1_o)

    return edge_a


CHB = 80
NCHB = EPW // CHB


@functools.lru_cache(maxsize=None)
def _edge_pass_b_kernel():
  @functools.partial(
    pl.kernel,
    out_type=[
        jax.ShapeDtypeStruct((N_NODES, HH), _f32),
        jax.ShapeDtypeStruct((N_NODES, HH), _f32),
    ],
    mesh=_mesh(),
    scratch_types=[
        pltpu.VMEM((NCHB, CHB), jnp.int32),
        pltpu.VMEM((2, CHB, HH), _f32),
        pltpu.VMEM_SHARED((N_NODES, HH), _f32),
        pltpu.SemaphoreType.DMA,
        pltpu.SemaphoreType.DMA,
    ],
  )
  def _edge_pass_b(dst3, zT0, zT1, zeros_h, d0_o, d1_o,
                   dst_all, zr2, acc_sh, sem_r, sem_s):
    cid = lax.axis_index("c")
    sid = lax.axis_index("s")

    def run(zT, d_o):
        _acc_rows_io(lambda s: zeros_h.at[s], lambda s: acc_sh.at[s], sid)
        pltpu.sync_copy(dst3.at[sid], dst_all)
        plsc.subcore_barrier()
        ebase = sid * EPW
        pltpu.async_copy(zT.at[pl.ds(ebase, CHB)], zr2.at[0], sem_r)

        def do_chunk(i, b):
            base = ebase + i * CHB
            pltpu.make_async_copy(zT.at[pl.ds(base, CHB)], zr2.at[b], sem_r).wait()

            @pl.when(i >= 1)
            def _():
                pltpu.make_async_copy(zT.at[pl.ds(base, CHB)], zr2.at[b], sem_s).wait()

            @pl.when(i + 1 < NCHB)
            def _():
                pltpu.async_copy(zT.at[pl.ds(base + CHB, CHB)], zr2.at[1 - b], sem_r)

            zr = zr2.at[b]

            def edge(j, __):
                for r in range(8):
                    sl = pl.ds(r * L, L)
                    zr[j, sl] = _sigmoid(zr[j, sl])
                return 0

            lax.fori_loop(0, CHB, edge, 0)
            pltpu.async_copy(zr, acc_sh.at[dst_all.at[i]], sem_s, add=True)

        def chunk_pair(g, _):
            for b in range(2):
                do_chunk(2 * g + b, b)
            return 0

        lax.fori_loop(0, NCHB // 2, chunk_pair, 0)
        if NCHB % 2:
            base = ebase + (NCHB - 1) * CHB
            pltpu.make_async_copy(zT.at[pl.ds(base, CHB)], zr2.at[0], sem_r).wait()
            pltpu.make_async_copy(zT.at[pl.ds(base, CHB)], zr2.at[0], sem_s).wait()
            zr = zr2.at[0]

            def edge_t(j, __):
                for r in range(8):
                    sl = pl.ds(r * L, L)
                    zr[j, sl] = _sigmoid(zr[j, sl])
                return 0

            lax.fori_loop(0, CHB, edge_t, 0)
            pltpu.async_copy(zr, acc_sh.at[dst_all.at[NCHB - 1]], sem_s, add=True)
        pltpu.make_async_copy(zT.at[pl.ds(ebase, CHB)], zr2.at[0], sem_s).wait()
        plsc.subcore_barrier()
        _acc_rows_io(lambda s: acc_sh.at[s], lambda s: d_o.at[s], sid)

    @pl.when(cid == 0)
    def _():
        run(zT0, d0_o)

    @pl.when(cid == 1)
    def _():
        run(zT1, d1_o)

  return _edge_pass_b


CHR = 40
NCHR = EPW // CHR


@functools.lru_cache(maxsize=None)
def _rel_pass_kernel():
  @functools.partial(
    pl.kernel,
    out_type=[
        jax.ShapeDtypeStruct((ETP, HH), _f32),
        jax.ShapeDtypeStruct((ETP, HH), _f32),
        jax.ShapeDtypeStruct((ETP, HH), _f32),
    ],
    mesh=_mesh(),
    scratch_types=[
        pltpu.VMEM((NCHR, CHR), jnp.int32),
        pltpu.VMEM((2, CHR, HH), _f32),
        pltpu.VMEM((2, CHR, HH), _f32),
        pltpu.VMEM((CHR, HH), _f32),
        pltpu.VMEM((CHR, HH), _f32),
        pltpu.VMEM((2, HH), _f32),
        pltpu.VMEM((2, HH), _f32),
        pltpu.VMEM_SHARED((ETP, HH), _f32),
        pltpu.VMEM_SHARED((ETP, HH), _f32),
        pltpu.SemaphoreType.DMA,
        pltpu.SemaphoreType.DMA,
        pltpu.SemaphoreType.DMA,
    ],
  )
  def _rel_pass(et3, z0T0, z0T1, z1T0, z1T1, b00, b01, b10, b11,
                zeros_e, rs0_o, rs1_o, cnt_o,
                et_all, z0r2, z1r2, r_v, ones_v, bn0_v, bn1_v,
                rsum_sh, cnt_sh, sem_r, sem_s, sem_c):
    cid = lax.axis_index("c")
    sid = lax.axis_index("s")

    def run(z0T, z1T, b0, b1, rs_o, count):
        pltpu.sync_copy(zeros_e.at[pl.ds(sid * ERW, ERW)],
                        rsum_sh.at[pl.ds(sid * ERW, ERW)])
        if count:
            pltpu.sync_copy(zeros_e.at[pl.ds(sid * ERW, ERW)],
                            cnt_sh.at[pl.ds(sid * ERW, ERW)])
        pltpu.sync_copy(b0, bn0_v)
        pltpu.sync_copy(b1, bn1_v)
        pltpu.sync_copy(et3.at[sid], et_all)

        if count:
            def fill(j, _):
                for r in range(8):
                    ones_v[j, pl.ds(r * L, L)] = jnp.full((L,), 1.0, _f32)
                return 0

            lax.fori_loop(0, CHR, fill, 0)
        plsc.subcore_barrier()
        ebase = sid * EPW
        pltpu.async_copy(z0T.at[pl.ds(ebase, CHR)], z0r2.at[0], sem_r)
        pltpu.async_copy(z1T.at[pl.ds(ebase, CHR)], z1r2.at[0], sem_r)

        def chunk_pair(g, _):
            for b in range(2):
                i = 2 * g + b
                base = ebase + i * CHR
                pltpu.make_async_copy(z0T.at[pl.ds(base, CHR)], z0r2.at[b], sem_r).wait()
                pltpu.make_async_copy(z1T.at[pl.ds(base, CHR)], z1r2.at[b], sem_r).wait()

                @pl.when(i >= 1)
                def _():
                    pltpu.make_async_copy(z0T.at[pl.ds(base, CHR)], r_v, sem_s).wait()
                    if count:
                        pltpu.make_async_copy(z0T.at[pl.ds(base, CHR)], ones_v, sem_c).wait()

                @pl.when(i + 1 < NCHR)
                def _():
                    pltpu.async_copy(z0T.at[pl.ds(base + CHR, CHR)], z0r2.at[1 - b], sem_r)
                    pltpu.async_copy(z1T.at[pl.ds(base + CHR, CHR)], z1r2.at[1 - b], sem_r)

                z0r = z0r2.at[b]
                z1r = z1r2.at[b]

                def edge(j, __):
                    for r in range(8):
                        sl = pl.ds(r * L, L)
                        a0 = bn0_v[0, sl]
                        c0 = bn0_v[1, sl]
                        a1 = bn1_v[0, sl]
                        c1 = bn1_v[1, sl]
                        r0 = jnp.maximum(a0 * z0r[j, sl] + c0, 0.0)
                        r1 = jnp.maximum(a1 * z1r[j, sl] + c1, 0.0)
                        r_v[j, sl] = r0 + r1
                    return 0

                lax.fori_loop(0, CHR, edge, 0)
                pltpu.async_copy(r_v, rsum_sh.at[et_all.at[i]], sem_s, add=True)
                if count:
                    pltpu.async_copy(ones_v, cnt_sh.at[et_all.at[i]], sem_c, add=True)
            return 0

        lax.fori_loop(0, NCHR // 2, chunk_pair, 0)
        pltpu.make_async_copy(z0T.at[pl.ds(ebase, CHR)], r_v, sem_s).wait()
        if count:
            pltpu.make_async_copy(z0T.at[pl.ds(ebase, CHR)], ones_v, sem_c).wait()
        plsc.subcore_barrier()
        pltpu.sync_copy(rsum_sh.at[pl.ds(sid * ERW, ERW)],
                        rs_o.at[pl.ds(sid * ERW, ERW)])
        if count:
            pltpu.sync_copy(cnt_sh.at[pl.ds(sid * ERW, ERW)],
                            cnt_o.at[pl.ds(sid * ERW, ERW)])

    @pl.when(cid == 0)
    def _():
        run(z0T0, z1T0, b00, b10, rs0_o, True)

    @pl.when(cid == 1)
    def _():
        run(z0T1, z1T1, b01, b11, rs1_o, False)

  return _rel_pass


_MB = 1000


def _dense_body(h_ref, w_ref, b_ref, a_ref, d0_ref, d1_ref, s0_ref, s1_ref):
    y = jnp.dot(h_ref[...], w_ref[...], preferred_element_type=_f32)
    y = y + b_ref[...]
    a_ref[...] = y[:, 0:256]
    d0_ref[...] = y[:, 256:384]
    d1_ref[...] = y[:, 384:512]
    s0_ref[...] = jnp.concatenate([y[:, 512:640], y[:, 768:896]], axis=1)
    s1_ref[...] = jnp.concatenate([y[:, 640:768], y[:, 896:1024]], axis=1)


def _dense(h, wcat, bcat):
    n = h.shape[0]
    grid = (n // _MB,)
    return pl.pallas_call(
        _dense_body,
        grid=grid,
        in_specs=[
            pl.BlockSpec((_MB, HID), lambda i: (i, 0)),
            pl.BlockSpec((HID, 4 * HID), lambda i: (0, 0)),
            pl.BlockSpec((1, 4 * HID), lambda i: (0, 0)),
        ],
        out_specs=[
            pl.BlockSpec((_MB, HID), lambda i: (i, 0)),
            pl.BlockSpec((_MB, HH), lambda i: (i, 0)),
            pl.BlockSpec((_MB, HH), lambda i: (i, 0)),
            pl.BlockSpec((_MB, HID), lambda i: (i, 0)),
            pl.BlockSpec((_MB, HID), lambda i: (i, 0)),
        ],
        out_shape=[
            jax.ShapeDtypeStruct((n, HID), _f32),
            jax.ShapeDtypeStruct((n, HH), _f32),
            jax.ShapeDtypeStruct((n, HH), _f32),
            jax.ShapeDtypeStruct((n, HID), _f32),
            jax.ShapeDtypeStruct((n, HID), _f32),
        ],
    )(h, wcat, bcat)


def _cet_body(e_ref, w0_ref, b0_ref, w1_ref, b1_ref, c00, c01, c10, c11):
    y0 = jnp.dot(e_ref[...], w0_ref[...], preferred_element_type=_f32) + b0_ref[...]
    y1 = jnp.dot(e_ref[...], w1_ref[...], preferred_element_type=_f32) + b1_ref[...]
    c00[...] = y0[:, :HH]
    c01[...] = y0[:, HH:]
    c10[...] = y1[:, :HH]
    c11[...] = y1[:, HH:]


def _cet(emb_e_pad, cw0, cb0, cw1, cb1):
    return pl.pallas_call(
        _cet_body,
        out_shape=[jax.ShapeDtypeStruct((ETP, HH), _f32)] * 4,
    )(emb_e_pad, cw0, cb0.reshape(1, HID), cw1, cb1.reshape(1, HID))


def _node_body(with_stats, *refs):
    if with_stats:
        (h_ref, a_ref, n0, n1, d0, d1, g_ref, b_ref, st0, st1,
         ge_ref, be_ref, out_ref, bn0_ref, bn1_ref) = refs
    else:
        (h_ref, a_ref, n0, n1, d0, d1, g_ref, b_ref, out_ref) = refs
    num = jnp.concatenate([n0[...], n1[...]], axis=1)
    den = jnp.concatenate([d0[...], d1[...]], axis=1)
    x = a_ref[...] + num / (den + 1e-6)
    mu = jnp.mean(x, axis=0, keepdims=True)
    var = jnp.mean((x - mu) ** 2, axis=0, keepdims=True)
    y = g_ref[...] * (x - mu) / jnp.sqrt(var + 1e-5) + b_ref[...]
    out_ref[...] = h_ref[...] + jnp.maximum(y, 0.0)
    if with_stats:
        s0 = jnp.sum(st0[...], axis=0)
        s1 = jnp.sum(st1[...], axis=0)
        for s, ref, lo in ((s0, bn0_ref, 0), (s1, bn1_ref, HH)):
            mean = s[0:1] / N_EDGES
            v = s[1:2] / N_EDGES - mean * mean
            aa = ge_ref[:, lo:lo + HH] / jnp.sqrt(v + 1e-5)
            cc = be_ref[:, lo:lo + HH] - aa * mean
            ref[...] = jnp.concatenate([aa, cc], axis=0)


def _node_update(h_in, a, n0, n1, d0, d1, g, b, stats=None, ge=None, be=None):
    n = h_in.shape[0]
    if stats is None:
        out_shape = jax.ShapeDtypeStruct((n, HID), _f32)
        args = (h_in, a, n0, n1, d0, d1, g.reshape(1, HID), b.reshape(1, HID))
        body = functools.partial(_node_body, False)
    else:
        out_shape = [
            jax.ShapeDtypeStruct((n, HID), _f32),
            jax.ShapeDtypeStruct((2, HH), _f32),
            jax.ShapeDtypeStruct((2, HH), _f32),
        ]
        args = (h_in, a, n0, n1, d0, d1, g.reshape(1, HID), b.reshape(1, HID),
                stats[0], stats[1], ge.reshape(1, HID), be.reshape(1, HID))
        body = functools.partial(_node_body, True)
    return pl.pallas_call(
        body,
        out_shape=out_shape,
        compiler_params=pltpu.CompilerParams(
            vmem_limit_bytes=110 * 1024 * 1024),
    )(*args)


_CMB = 1600


def _cm_body(z0_ref, z1_ref, st0_ref, st1_ref, cw_ref, ge_ref, be_ref,
             x0_ref, x1_ref, bn0_ref, bn1_ref):
    s0 = jnp.sum(st0_ref[...], axis=0)
    s1 = jnp.sum(st1_ref[...], axis=0)
    mean = jnp.concatenate([s0[0:1], s1[0:1]], axis=1) / N_EDGES
    var = jnp.concatenate([s0[1:2], s1[1:2]], axis=1) / N_EDGES - mean * mean
    a = ge_ref[...] / jnp.sqrt(var + 1e-5)
    c = be_ref[...] - a * mean
    z = jnp.concatenate([z0_ref[...], z1_ref[...]], axis=1)
    r = jnp.maximum(a * z + c, 0.0)
    x = jnp.dot(r, cw_ref[...], preferred_element_type=_f32)
    x0_ref[...] = x[:, :HH]
    x1_ref[...] = x[:, HH:]
    bn0_ref[...] = jnp.concatenate([a[:, :HH], c[:, :HH]], axis=0)
    bn1_ref[...] = jnp.concatenate([a[:, HH:], c[:, HH:]], axis=0)


def _cmatmul(z0, z1, st0, st1, cw, ge, be):
    grid = (N_EDGES // _CMB,)
    return pl.pallas_call(
        _cm_body,
        grid=grid,
        in_specs=[
            pl.BlockSpec((_CMB, HH), lambda i: (i, 0)),
            pl.BlockSpec((_CMB, HH), lambda i: (i, 0)),
            pl.BlockSpec((NS, 2, HH), lambda i: (0, 0, 0)),
            pl.BlockSpec((NS, 2, HH), lambda i: (0, 0, 0)),
            pl.BlockSpec((HID, HID), lambda i: (0, 0)),
            pl.BlockSpec((1, HID), lambda i: (0, 0)),
            pl.BlockSpec((1, HID), lambda i: (0, 0)),
        ],
        out_specs=[
            pl.BlockSpec((_CMB, HH), lambda i: (i, 0)),
            pl.BlockSpec((_CMB, HH), lambda i: (i, 0)),
            pl.BlockSpec((2, HH), lambda i: (0, 0)),
            pl.BlockSpec((2, HH), lambda i: (0, 0)),
        ],
        out_shape=[
            jax.ShapeDtypeStruct((N_EDGES, HH), _f32),
            jax.ShapeDtypeStruct((N_EDGES, HH), _f32),
            jax.ShapeDtypeStruct((2, HH), _f32),
            jax.ShapeDtypeStruct((2, HH), _f32),
        ],
    )(z0, z1, st0, st1, cw, ge.reshape(1, HID), be.reshape(1, HID))


def _rel_body(r0_ref, r1_ref, cnt_ref, emb_ref, out_ref):
    cnt = cnt_ref[:, 0:1]
    rsum = jnp.concatenate([r0_ref[...], r1_ref[...]], axis=1)
    total = cnt * emb_ref[...] + rsum
    out_ref[...] = total / jnp.maximum(cnt, 1.0)


def _rel_means(r0, r1, cnt, emb_e_pad):
    return pl.pallas_call(
        _rel_body,
        out_shape=jax.ShapeDtypeStruct((ETP, HID), _f32),
    )(r0, r1, cnt, emb_e_pad)


def kernel(node_id, edge_index, edge_type, p, emb_h, emb_e,
           Aw, Ab, Bw, Bb, Cw, Cb, Dw, Db, Ew, Eb, gh, bh, ge, be):
    src = edge_index[0].astype(jnp.int32)
    dst = edge_index[1].astype(jnp.int32)
    et = edge_type.astype(jnp.int32)

    nid_pad = jnp.concatenate(
        [node_id.astype(jnp.int32), jnp.zeros((NID_PAD - N_NODES,), jnp.int32)])
    emb_e_pad = jnp.concatenate(
        [emb_e, jnp.zeros((ETP - emb_e.shape[0], HID), _f32)], axis=0)

    zeros_n = jnp.zeros((N_NODES, HH), _f32)
    dst3 = dst.reshape(NS, NCHB, CHB)
    et3 = et.reshape(NS, NCHR, CHR)
    zeros_e = jnp.zeros((ETP, HH), _f32)

    h0 = _gather_h0_kernel()(emb_h, nid_pad)[:N_NODES]

    cet = _cet(emb_e_pad, Cw[0], Cb[0], Cw[1], Cb[1])

    def wcat(l):
        w = jnp.concatenate([Aw[l], Ew[l], Dw[l], Bw[l]], axis=1)
        bb = jnp.concatenate([Ab[l], Eb[l], Db[l], Bb[l]]).reshape(1, 4 * HID)
        return w, bb

    w0, b0 = wcat(0)
    a0, e00, e01, s00, s01 = _dense(h0, w0, b0)
    z00, z01, n00, n01, st00, st01 = _make_edge_pass_a(False)(
        src, dst, et, s00, s01, e00, e01, cet[0], cet[1], zeros_n)
    d00, d01 = _edge_pass_b_kernel()(dst3, z00, z01, zeros_n)
    h1 = _node_update(h0, a0, n00, n01, d00, d01, gh[0], bh[0])

    w1, b1 = wcat(1)
    a1, e10, e11, s10, s11 = _dense(h1, w1, b1)
    x0, x1, bn00, bn01 = _cmatmul(z00, z01, st00, st01, Cw[1], ge[0], be[0])
    z10, z11, n10, n11, st10, st11 = _make_edge_pass_a(True)(
        src, dst, et, s10, s11, e10, e11, cet[2], cet[3], x0, x1, zeros_n)
    d10, d11 = _edge_pass_b_kernel()(dst3, z10, z11, zeros_n)
    h2, bn10, bn11 = _node_update(h1, a1, n10, n11, d10, d11, gh[1], bh[1],
                                  stats=(st10, st11), ge=ge[1], be=be[1])

    rs0, rs1, cnt = _rel_pass_kernel()(et3, z00, z01, z10, z11,
                                       bn00, bn01, bn10, bn11, zeros_e)
    rel = _rel_means(rs0, rs1, cnt, emb_e_pad)[:emb_e.shape[0]]
    return (h2, rel)


# speedup vs baseline: 1.4230x; 1.0987x over previous
import functools

import jax
import jax.numpy as jnp
from jax import lax
from jax.experimental import pallas as pl
from jax.experimental.pallas import tpu as pltpu
from jax.experimental.pallas import tpu_sc as plsc

N_NODES = 10000
N_EDGES = 160000
HID = 256
HH = 128
ETP = 512
NC, NS, L = 2, 16, 16
NW = NC * NS

EPW = N_EDGES // NS
CH = 40
NCH = EPW // CH
NPW = N_NODES // NS
ERW = ETP // NS

_f32 = jnp.float32
@functools.lru_cache(maxsize=None)
def _mesh():
    return plsc.VectorSubcoreMesh(core_axis_name="c", subcore_axis_name="s")


def _sigmoid(z):
    return 1.0 / (1.0 + jnp.exp(-z))


_NR8 = 624
_NREM = N_NODES - NS * _NR8


def _acc_rows_io(src_at, dst_at, sid):
    pltpu.sync_copy(src_at(pl.ds(sid * _NR8, _NR8)),
                    dst_at(pl.ds(sid * _NR8, _NR8)))

    @pl.when(sid == NS - 1)
    def _():
        pltpu.sync_copy(src_at(pl.ds(NS * _NR8, _NREM)),
                        dst_at(pl.ds(NS * _NR8, _NREM)))


NID_PAD = 10240
_GPW = NID_PAD // NW


@functools.lru_cache(maxsize=None)
def _gather_h0_kernel():
    @functools.partial(
        pl.kernel,
        out_type=jax.ShapeDtypeStruct((NID_PAD, HID), _f32),
        mesh=_mesh(),
        scratch_types=[
            pltpu.VMEM((_GPW,), jnp.int32),
            pltpu.VMEM((_GPW, HID), _f32),
            pltpu.SemaphoreType.DMA,
        ],
    )
    def _gather_h0(table, nid, out, idx_v, rows_v, sem):
        wid = lax.axis_index("s") * NC + lax.axis_index("c")
        base = wid * _GPW
        pltpu.sync_copy(nid.at[pl.ds(base, _GPW)], idx_v)
        pltpu.async_copy(table.at[idx_v], rows_v, sem).wait()
        pltpu.sync_copy(rows_v, out.at[pl.ds(base, _GPW)])

    return _gather_h0


@functools.lru_cache(maxsize=None)
def _make_edge_pass_a(with_x):
    n_x = 2 if with_x else 0
    scratch = [
        pltpu.VMEM((2, CH), jnp.int32),
        pltpu.VMEM((2, CH), jnp.int32),
        pltpu.VMEM((2, CH), jnp.int32),
        pltpu.VMEM((2, CH), jnp.int32),
        pltpu.VMEM((2, CH, HID), _f32),
        pltpu.VMEM((2, CH, HH), _f32),
        pltpu.VMEM((2, CH, HH), _f32),
        pltpu.VMEM((CH, HH), _f32),
        pltpu.VMEM((2, HH), _f32),
        pltpu.VMEM_SHARED((N_NODES, HH), _f32),
        pltpu.SemaphoreType.DMA,
        pltpu.SemaphoreType.DMA,
        pltpu.SemaphoreType.DMA,
        pltpu.SemaphoreType.DMA,
        pltpu.SemaphoreType.DMA,
        pltpu.SemaphoreType.DMA,
    ]
    out_type = [
        jax.ShapeDtypeStruct((N_EDGES, HH), _f32),
        jax.ShapeDtypeStruct((N_EDGES, HH), _f32),
        jax.ShapeDtypeStruct((N_NODES, HH), _f32),
        jax.ShapeDtypeStruct((N_NODES, HH), _f32),
        jax.ShapeDtypeStruct((NS, 2, HH), _f32),
        jax.ShapeDtypeStruct((NS, 2, HH), _f32),
    ]

    @functools.partial(pl.kernel, out_type=out_type, mesh=_mesh(),
                       scratch_types=scratch)
    def edge_a(*refs):
        (src_h, dst_h, et_h, sT0, sT1, dT0, dT1, cT0, cT1) = refs[:9]
        xs = refs[9:9 + n_x]
        zeros_h = refs[9 + n_x]
        (z0_o, z1_o, n0_o, n1_o, st0_o, st1_o) = refs[10 + n_x:16 + n_x]
        (si2, di2, ei2, dsc2, srows2, drows2, crows2, xrows,
         stats_v, acc_sh, sem_i, sem_g, sem_x, sem_z, sem_s, sem_d) = refs[16 + n_x:]
        cid = lax.axis_index("c")
        sid = lax.axis_index("s")

        def run(sT, dT, cT, xT, z_o, n_o, st_o):
            _acc_rows_io(lambda s: zeros_h.at[s], lambda s: acc_sh.at[s], sid)
            plsc.subcore_barrier()
            ebase = sid * EPW

            def fire_idx(base, b):
                pltpu.async_copy(src_h.at[pl.ds(base, CH)], si2.at[b], sem_i)
                pltpu.async_copy(dst_h.at[pl.ds(base, CH)], di2.at[b], sem_i)
                pltpu.async_copy(et_h.at[pl.ds(base, CH)], ei2.at[b], sem_i)

            def drain_idx(base, b):
                pltpu.make_async_copy(src_h.at[pl.ds(base, CH)], si2.at[b], sem_i).wait()
                pltpu.make_async_copy(dst_h.at[pl.ds(base, CH)], di2.at[b], sem_i).wait()
                pltpu.make_async_copy(et_h.at[pl.ds(base, CH)], ei2.at[b], sem_i).wait()

            def fire_gathers(b):
                pltpu.async_copy(sT.at[si2.at[b]], srows2.at[b], sem_g)
                pltpu.async_copy(dT.at[di2.at[b]], drows2.at[b], sem_g)
                pltpu.async_copy(cT.at[ei2.at[b]], crows2.at[b], sem_g)

            def drain_gathers(b):
                pltpu.make_async_copy(sT.at[si2.at[b]], srows2.at[b], sem_g).wait()
                pltpu.make_async_copy(dT.at[di2.at[b]], drows2.at[b], sem_g).wait()
                pltpu.make_async_copy(cT.at[ei2.at[b]], crows2.at[b], sem_g).wait()

            fire_idx(ebase, 0)
            drain_idx(ebase, 0)
            fire_gathers(0)
            if with_x:
                pltpu.async_copy(xT.at[pl.ds(ebase, CH)], xrows, sem_x)
            fire_idx(ebase + CH, 1)
            pltpu.async_copy(dst_h.at[pl.ds(ebase, CH)], dsc2.at[0], sem_d)

            def chunk_pair(g, carry):
                for b in range(2):
                    i = 2 * g + b
                    base = ebase + i * CH
                    drain_gathers(b)
                    if with_x:
                        pltpu.make_async_copy(xT.at[pl.ds(base, CH)], xrows, sem_x).wait()

                    @pl.when(i >= 1)
                    def _():
                        pltpu.make_async_copy(crows2.at[b], z_o.at[pl.ds(base, CH)], sem_z).wait()
                        pltpu.make_async_copy(z_o.at[pl.ds(base, CH)], drows2.at[b], sem_s).wait()

                    @pl.when(i + 1 < NCH)
                    def _():
                        drain_idx(ebase + (i + 1) * CH, 1 - b)
                        fire_gathers(1 - b)

                    @pl.when(i + 2 < NCH)
                    def _():
                        fire_idx(ebase + (i + 2) * CH, b)

                    sr = srows2.at[b]
                    dr = drows2.at[b]
                    cr = crows2.at[b]

                    def edge(jp, car):
                        acc = list(car)
                        for jj in range(2):
                            j = jp * 2 + jj
                            for r in range(8):
                                sl = pl.ds(r * L, L)
                                z = sr[j, sl] + dr[j, sl] + cr[j, sl]
                                if with_x:
                                    z = z + xrows[j, sl]
                                sig = _sigmoid(z)
                                cr[j, sl] = z
                                dr[j, sl] = sig * sr[j, pl.ds(HH + r * L, L)]
                                acc[2 * r] = acc[2 * r] + z
                                acc[2 * r + 1] = acc[2 * r + 1] + z * z
                        return tuple(acc)

                    carry = lax.fori_loop(0, CH // 2, edge, carry)
                    pltpu.async_copy(cr, z_o.at[pl.ds(base, CH)], sem_z)
                    pltpu.make_async_copy(dst_h.at[pl.ds(base, CH)], dsc2.at[b], sem_d).wait()
                    pltpu.async_copy(dr, acc_sh.at[dsc2.at[b]], sem_s, add=True)

                    @pl.when(i + 1 < NCH)
                    def _():
                        pltpu.async_copy(dst_h.at[pl.ds(base + CH, CH)], dsc2.at[1 - b], sem_d)
                        if with_x:
                            pltpu.async_copy(xT.at[pl.ds(base + CH, CH)], xrows, sem_x)
                return carry

            zero = jnp.zeros((L,), _f32)
            carry = lax.fori_loop(0, NCH // 2, chunk_pair,
                                  tuple(zero for _ in range(16)))
            pltpu.make_async_copy(crows2.at[0], z_o.at[pl.ds(ebase, CH)], sem_z).wait()
            pltpu.make_async_copy(z_o.at[pl.ds(ebase, CH)], drows2.at[0], sem_s).wait()
            for r in range(8):
                stats_v[0, pl.ds(r * L, L)] = carry[2 * r]
                stats_v[1, pl.ds(r * L, L)] = carry[2 * r + 1]
            pltpu.sync_copy(stats_v, st_o.at[sid])
            plsc.subcore_barrier()
            _acc_rows_io(lambda s: acc_sh.at[s], lambda s: n_o.at[s], sid)

        x0 = xs[0] if with_x else None
        x1 = xs[1] if with_x else None

        @pl.when(cid == 0)
        def _():
            run(sT0, dT0, cT0, x0, z0_o, n0_o, st0_o)

        @pl.when(cid == 1)
        def _():
            run(sT1, dT1, cT1, x1, z1_o, n1_o, st1_o)

    return edge_a


CHB = 80
NCHB = EPW // CHB


@functools.lru_cache(maxsize=None)
def _edge_pass_b_kernel():
  @functools.partial(
    pl.kernel,
    out_type=[
        jax.ShapeDtypeStruct((N_NODES, HH), _f32),
        jax.ShapeDtypeStruct((N_NODES, HH), _f32),
    ],
    mesh=_mesh(),
    scratch_types=[
        pltpu.VMEM((NCHB, CHB), jnp.int32),
        pltpu.VMEM((2, CHB, HH), _f32),
        pltpu.VMEM_SHARED((N_NODES, HH), _f32),
        pltpu.SemaphoreType.DMA,
        pltpu.SemaphoreType.DMA,
    ],
  )
  def _edge_pass_b(dst3, zT0, zT1, zeros_h, d0_o, d1_o,
                   dst_all, zr2, acc_sh, sem_r, sem_s):
    cid = lax.axis_index("c")
    sid = lax.axis_index("s")

    def run(zT, d_o):
        _acc_rows_io(lambda s: zeros_h.at[s], lambda s: acc_sh.at[s], sid)
        pltpu.sync_copy(dst3.at[sid], dst_all)
        plsc.subcore_barrier()
        ebase = sid * EPW
        pltpu.async_copy(zT.at[pl.ds(ebase, CHB)], zr2.at[0], sem_r)

        def do_chunk(i, b):
            base = ebase + i * CHB
            pltpu.make_async_copy(zT.at[pl.ds(base, CHB)], zr2.at[b], sem_r).wait()

            @pl.when(i >= 1)
            def _():
                pltpu.make_async_copy(zT.at[pl.ds(base, CHB)], zr2.at[b], sem_s).wait()

            @pl.when(i + 1 < NCHB)
            def _():
                pltpu.async_copy(zT.at[pl.ds(base + CHB, CHB)], zr2.at[1 - b], sem_r)

            zr = zr2.at[b]

            def edge(j, __):
                for r in range(8):
                    sl = pl.ds(r * L, L)
                    zr[j, sl] = _sigmoid(zr[j, sl])
                return 0

            lax.fori_loop(0, CHB, edge, 0)
            pltpu.async_copy(zr, acc_sh.at[dst_all.at[i]], sem_s, add=True)

        def chunk_pair(g, _):
            for b in range(2):
                do_chunk(2 * g + b, b)
            return 0

        lax.fori_loop(0, NCHB // 2, chunk_pair, 0)
        if NCHB % 2:
            base = ebase + (NCHB - 1) * CHB
            pltpu.make_async_copy(zT.at[pl.ds(base, CHB)], zr2.at[0], sem_r).wait()
            pltpu.make_async_copy(zT.at[pl.ds(base, CHB)], zr2.at[0], sem_s).wait()
            zr = zr2.at[0]

            def edge_t(j, __):
                for r in range(8):
                    sl = pl.ds(r * L, L)
                    zr[j, sl] = _sigmoid(zr[j, sl])
                return 0

            lax.fori_loop(0, CHB, edge_t, 0)
            pltpu.async_copy(zr, acc_sh.at[dst_all.at[NCHB - 1]], sem_s, add=True)
        pltpu.make_async_copy(zT.at[pl.ds(ebase, CHB)], zr2.at[0], sem_s).wait()
        plsc.subcore_barrier()
        _acc_rows_io(lambda s: acc_sh.at[s], lambda s: d_o.at[s], sid)

    @pl.when(cid == 0)
    def _():
        run(zT0, d0_o)

    @pl.when(cid == 1)
    def _():
        run(zT1, d1_o)

  return _edge_pass_b


CHR = 40
NCHR = EPW // CHR


@functools.lru_cache(maxsize=None)
def _rel_pass_kernel():
  @functools.partial(
    pl.kernel,
    out_type=[
        jax.ShapeDtypeStruct((ETP, HH), _f32),
        jax.ShapeDtypeStruct((ETP, HH), _f32),
        jax.ShapeDtypeStruct((ETP, HH), _f32),
    ],
    mesh=_mesh(),
    scratch_types=[
        pltpu.VMEM((NCHR, CHR), jnp.int32),
        pltpu.VMEM((2, CHR, HH), _f32),
        pltpu.VMEM((2, CHR, HH), _f32),
        pltpu.VMEM((CHR, HH), _f32),
        pltpu.VMEM((CHR, HH), _f32),
        pltpu.VMEM((2, HH), _f32),
        pltpu.VMEM((2, HH), _f32),
        pltpu.VMEM_SHARED((ETP, HH), _f32),
        pltpu.VMEM_SHARED((ETP, HH), _f32),
        pltpu.SemaphoreType.DMA,
        pltpu.SemaphoreType.DMA,
        pltpu.SemaphoreType.DMA,
    ],
  )
  def _rel_pass(et3, z0T0, z0T1, z1T0, z1T1, b00, b01, b10, b11,
                zeros_e, rs0_o, rs1_o, cnt_o,
                et_all, z0r2, z1r2, r_v, ones_v, bn0_v, bn1_v,
                rsum_sh, cnt_sh, sem_r, sem_s, sem_c):
    cid = lax.axis_index("c")
    sid = lax.axis_index("s")

    def run(z0T, z1T, b0, b1, rs_o, count):
        pltpu.sync_copy(zeros_e.at[pl.ds(sid * ERW, ERW)],
                        rsum_sh.at[pl.ds(sid * ERW, ERW)])
        if count:
            pltpu.sync_copy(zeros_e.at[pl.ds(sid * ERW, ERW)],
                            cnt_sh.at[pl.ds(sid * ERW, ERW)])
        pltpu.sync_copy(b0, bn0_v)
        pltpu.sync_copy(b1, bn1_v)
        pltpu.sync_copy(et3.at[sid], et_all)

        if count:
            def fill(j, _):
                for r in range(8):
                    ones_v[j, pl.ds(r * L, L)] = jnp.full((L,), 1.0, _f32)
                return 0

            lax.fori_loop(0, CHR, fill, 0)
        plsc.subcore_barrier()
        ebase = sid * EPW
        pltpu.async_copy(z0T.at[pl.ds(ebase, CHR)], z0r2.at[0], sem_r)
        pltpu.async_copy(z1T.at[pl.ds(ebase, CHR)], z1r2.at[0], sem_r)

        def chunk_pair(g, _):
            for b in range(2):
                i = 2 * g + b
                base = ebase + i * CHR
                pltpu.make_async_copy(z0T.at[pl.ds(base, CHR)], z0r2.at[b], sem_r).wait()
                pltpu.make_async_copy(z1T.at[pl.ds(base, CHR)], z1r2.at[b], sem_r).wait()

                @pl.when(i >= 1)
                def _():
                    pltpu.make_async_copy(z0T.at[pl.ds(base, CHR)], r_v, sem_s).wait()
                    if count:
                        pltpu.make_async_copy(z0T.at[pl.ds(base, CHR)], ones_v, sem_c).wait()

                @pl.when(i + 1 < NCHR)
                def _():
                    pltpu.async_copy(z0T.at[pl.ds(base + CHR, CHR)], z0r2.at[1 - b], sem_r)
                    pltpu.async_copy(z1T.at[pl.ds(base + CHR, CHR)], z1r2.at[1 - b], sem_r)

                z0r = z0r2.at[b]
                z1r = z1r2.at[b]

                def edge(j, __):
                    for r in range(8):
                        sl = pl.ds(r * L, L)
                        a0 = bn0_v[0, sl]
                        c0 = bn0_v[1, sl]
                        a1 = bn1_v[0, sl]
                        c1 = bn1_v[1, sl]
                        r0 = jnp.maximum(a0 * z0r[j, sl] + c0, 0.0)
                        r1 = jnp.maximum(a1 * z1r[j, sl] + c1, 0.0)
                        r_v[j, sl] = r0 + r1
                    return 0

                lax.fori_loop(0, CHR, edge, 0)
                pltpu.async_copy(r_v, rsum_sh.at[et_all.at[i]], sem_s, add=True)
                if count:
                    pltpu.async_copy(ones_v, cnt_sh.at[et_all.at[i]], sem_c, add=True)
            return 0

        lax.fori_loop(0, NCHR // 2, chunk_pair, 0)
        pltpu.make_async_copy(z0T.at[pl.ds(ebase, CHR)], r_v, sem_s).wait()
        if count:
            pltpu.make_async_copy(z0T.at[pl.ds(ebase, CHR)], ones_v, sem_c).wait()
        plsc.subcore_barrier()
        pltpu.sync_copy(rsum_sh.at[pl.ds(sid * ERW, ERW)],
                        rs_o.at[pl.ds(sid * ERW, ERW)])
        if count:
            pltpu.sync_copy(cnt_sh.at[pl.ds(sid * ERW, ERW)],
                            cnt_o.at[pl.ds(sid * ERW, ERW)])

    @pl.when(cid == 0)
    def _():
        run(z0T0, z1T0, b00, b10, rs0_o, True)

    @pl.when(cid == 1)
    def _():
        run(z0T1, z1T1, b01, b11, rs1_o, False)

  return _rel_pass


_MB = 1000


def _dense_body(h_ref, w_ref, b_ref, a_ref, d0_ref, d1_ref, s0_ref, s1_ref):
    y = jnp.dot(h_ref[...], w_ref[...], preferred_element_type=_f32)
    y = y + b_ref[...]
    a_ref[...] = y[:, 0:256]
    d0_ref[...] = y[:, 256:384]
    d1_ref[...] = y[:, 384:512]
    s0_ref[...] = jnp.concatenate([y[:, 512:640], y[:, 768:896]], axis=1)
    s1_ref[...] = jnp.concatenate([y[:, 640:768], y[:, 896:1024]], axis=1)


def _dense(h, wcat, bcat):
    n = h.shape[0]
    grid = (n // _MB,)
    return pl.pallas_call(
        _dense_body,
        grid=grid,
        in_specs=[
            pl.BlockSpec((_MB, HID), lambda i: (i, 0)),
            pl.BlockSpec((HID, 4 * HID), lambda i: (0, 0)),
            pl.BlockSpec((1, 4 * HID), lambda i: (0, 0)),
        ],
        out_specs=[
            pl.BlockSpec((_MB, HID), lambda i: (i, 0)),
            pl.BlockSpec((_MB, HH), lambda i: (i, 0)),
            pl.BlockSpec((_MB, HH), lambda i: (i, 0)),
            pl.BlockSpec((_MB, HID), lambda i: (i, 0)),
            pl.BlockSpec((_MB, HID), lambda i: (i, 0)),
        ],
        out_shape=[
            jax.ShapeDtypeStruct((n, HID), _f32),
            jax.ShapeDtypeStruct((n, HH), _f32),
            jax.ShapeDtypeStruct((n, HH), _f32),
            jax.ShapeDtypeStruct((n, HID), _f32),
            jax.ShapeDtypeStruct((n, HID), _f32),
        ],
    )(h, wcat, bcat)


def _cet_body(e_ref, w0_ref, b0_ref, w1_ref, b1_ref, c00, c01, c10, c11):
    y0 = jnp.dot(e_ref[...], w0_ref[...], preferred_element_type=_f32) + b0_ref[...]
    y1 = jnp.dot(e_ref[...], w1_ref[...], preferred_element_type=_f32) + b1_ref[...]
    c00[...] = y0[:, :HH]
    c01[...] = y0[:, HH:]
    c10[...] = y1[:, :HH]
    c11[...] = y1[:, HH:]


def _cet(emb_e_pad, cw0, cb0, cw1, cb1):
    return pl.pallas_call(
        _cet_body,
        out_shape=[jax.ShapeDtypeStruct((ETP, HH), _f32)] * 4,
    )(emb_e_pad, cw0, cb0.reshape(1, HID), cw1, cb1.reshape(1, HID))


def _node_body(with_stats, *refs):
    if with_stats:
        (h_ref, a_ref, n0, n1, d0, d1, g_ref, b_ref, st0, st1,
         ge_ref, be_ref, out_ref, bn0_ref, bn1_ref) = refs
    else:
        (h_ref, a_ref, n0, n1, d0, d1, g_ref, b_ref, out_ref) = refs
    num = jnp.concatenate([n0[...], n1[...]], axis=1)
    den = jnp.concatenate([d0[...], d1[...]], axis=1)
    x = a_ref[...] + num / (den + 1e-6)
    mu = jnp.mean(x, axis=0, keepdims=True)
    var = jnp.mean((x - mu) ** 2, axis=0, keepdims=True)
    y = g_ref[...] * (x - mu) / jnp.sqrt(var + 1e-5) + b_ref[...]
    out_ref[...] = h_ref[...] + jnp.maximum(y, 0.0)
    if with_stats:
        s0 = jnp.sum(st0[...], axis=0)
        s1 = jnp.sum(st1[...], axis=0)
        for s, ref, lo in ((s0, bn0_ref, 0), (s1, bn1_ref, HH)):
            mean = s[0:1] / N_EDGES
            v = s[1:2] / N_EDGES - mean * mean
            aa = ge_ref[:, lo:lo + HH] / jnp.sqrt(v + 1e-5)
            cc = be_ref[:, lo:lo + HH] - aa * mean
            ref[...] = jnp.concatenate([aa, cc], axis=0)


def _node_update(h_in, a, n0, n1, d0, d1, g, b, stats=None, ge=None, be=None):
    n = h_in.shape[0]
    if stats is None:
        out_shape = jax.ShapeDtypeStruct((n, HID), _f32)
        args = (h_in, a, n0, n1, d0, d1, g.reshape(1, HID), b.reshape(1, HID))
        body = functools.partial(_node_body, False)
    else:
        out_shape = [
            jax.ShapeDtypeStruct((n, HID), _f32),
            jax.ShapeDtypeStruct((2, HH), _f32),
            jax.ShapeDtypeStruct((2, HH), _f32),
        ]
        args = (h_in, a, n0, n1, d0, d1, g.reshape(1, HID), b.reshape(1, HID),
                stats[0], stats[1], ge.reshape(1, HID), be.reshape(1, HID))
        body = functools.partial(_node_body, True)
    return pl.pallas_call(
        body,
        out_shape=out_shape,
        compiler_params=pltpu.CompilerParams(
            vmem_limit_bytes=110 * 1024 * 1024),
    )(*args)


_CMB = 1600


def _cm_body(z0_ref, z1_ref, st0_ref, st1_ref, cw_ref, ge_ref, be_ref,
             x0_ref, x1_ref, bn0_ref, bn1_ref):
    s0 = jnp.sum(st0_ref[...], axis=0)
    s1 = jnp.sum(st1_ref[...], axis=0)
    mean = jnp.concatenate([s0[0:1], s1[0:1]], axis=1) / N_EDGES
    var = jnp.concatenate([s0[1:2], s1[1:2]], axis=1) / N_EDGES - mean * mean
    a = ge_ref[...] / jnp.sqrt(var + 1e-5)
    c = be_ref[...] - a * mean
    z = jnp.concatenate([z0_ref[...], z1_ref[...]], axis=1)
    r = jnp.maximum(a * z + c, 0.0)
    x = jnp.dot(r, cw_ref[...], preferred_element_type=_f32)
    x0_ref[...] = x[:, :HH]
    x1_ref[...] = x[:, HH:]
    bn0_ref[...] = jnp.concatenate([a[:, :HH], c[:, :HH]], axis=0)
    bn1_ref[...] = jnp.concatenate([a[:, HH:], c[:, HH:]], axis=0)


def _cmatmul(z0, z1, st0, st1, cw, ge, be):
    grid = (N_EDGES // _CMB,)
    return pl.pallas_call(
        _cm_body,
        grid=grid,
        in_specs=[
            pl.BlockSpec((_CMB, HH), lambda i: (i, 0)),
            pl.BlockSpec((_CMB, HH), lambda i: (i, 0)),
            pl.BlockSpec((NS, 2, HH), lambda i: (0, 0, 0)),
            pl.BlockSpec((NS, 2, HH), lambda i: (0, 0, 0)),
            pl.BlockSpec((HID, HID), lambda i: (0, 0)),
            pl.BlockSpec((1, HID), lambda i: (0, 0)),
            pl.BlockSpec((1, HID), lambda i: (0, 0)),
        ],
        out_specs=[
            pl.BlockSpec((_CMB, HH), lambda i: (i, 0)),
            pl.BlockSpec((_CMB, HH), lambda i: (i, 0)),
            pl.BlockSpec((2, HH), lambda i: (0, 0)),
            pl.BlockSpec((2, HH), lambda i: (0, 0)),
        ],
        out_shape=[
            jax.ShapeDtypeStruct((N_EDGES, HH), _f32),
            jax.ShapeDtypeStruct((N_EDGES, HH), _f32),
            jax.ShapeDtypeStruct((2, HH), _f32),
            jax.ShapeDtypeStruct((2, HH), _f32),
        ],
    )(z0, z1, st0, st1, cw, ge.reshape(1, HID), be.reshape(1, HID))


def _rel_body(r0_ref, r1_ref, cnt_ref, emb_ref, out_ref):
    cnt = cnt_ref[:, 0:1]
    rsum = jnp.concatenate([r0_ref[...], r1_ref[...]], axis=1)
    total = cnt * emb_ref[...] + rsum
    out_ref[...] = total / jnp.maximum(cnt, 1.0)


def _rel_means(r0, r1, cnt, emb_e_pad):
    return pl.pallas_call(
        _rel_body,
        out_shape=jax.ShapeDtypeStruct((ETP, HID), _f32),
    )(r0, r1, cnt, emb_e_pad)


def kernel(node_id, edge_index, edge_type, p, emb_h, emb_e,
           Aw, Ab, Bw, Bb, Cw, Cb, Dw, Db, Ew, Eb, gh, bh, ge, be):
    src = edge_index[0].astype(jnp.int32)
    dst = edge_index[1].astype(jnp.int32)
    et = edge_type.astype(jnp.int32)

    nid_pad = jnp.concatenate(
        [node_id.astype(jnp.int32), jnp.zeros((NID_PAD - N_NODES,), jnp.int32)])
    emb_e_pad = jnp.concatenate(
        [emb_e, jnp.zeros((ETP - emb_e.shape[0], HID), _f32)], axis=0)

    zeros_n = jnp.zeros((N_NODES, HH), _f32)
    dst3 = dst.reshape(NS, NCHB, CHB)
    et3 = et.reshape(NS, NCHR, CHR)
    zeros_e = jnp.zeros((ETP, HH), _f32)

    h0 = _gather_h0_kernel()(emb_h, nid_pad)[:N_NODES]

    cet = _cet(emb_e_pad, Cw[0], Cb[0], Cw[1], Cb[1])

    def wcat(l):
        w = jnp.concatenate([Aw[l], Ew[l], Dw[l], Bw[l]], axis=1)
        bb = jnp.concatenate([Ab[l], Eb[l], Db[l], Bb[l]]).reshape(1, 4 * HID)
        return w, bb

    w0, b0 = wcat(0)
    a0, e00, e01, s00, s01 = _dense(h0, w0, b0)
    z00, z01, n00, n01, st00, st01 = _make_edge_pass_a(False)(
        src, dst, et, s00, s01, e00, e01, cet[0], cet[1], zeros_n)
    d00, d01 = _edge_pass_b_kernel()(dst3, z00, z01, zeros_n)
    h1 = _node_update(h0, a0, n00, n01, d00, d01, gh[0], bh[0])

    w1, b1 = wcat(1)
    a1, e10, e11, s10, s11 = _dense(h1, w1, b1)
    x0, x1, bn00, bn01 = _cmatmul(z00, z01, st00, st01, Cw[1], ge[0], be[0])
    z10, z11, n10, n11, st10, st11 = _make_edge_pass_a(True)(
        src, dst, et, s10, s11, e10, e11, cet[2], cet[3], x0, x1, zeros_n)
    d10, d11 = _edge_pass_b_kernel()(dst3, z10, z11, zeros_n)
    h2, bn10, bn11 = _node_update(h1, a1, n10, n11, d10, d11, gh[1], bh[1],
                                  stats=(st10, st11), ge=ge[1], be=be[1])

    rs0, rs1, cnt = _rel_pass_kernel()(et3, z00, z01, z10, z11,
                                       bn00, bn01, bn10, bn11, zeros_e)
    rel = _rel_means(rs0, rs1, cnt, emb_e_pad)[:emb_e.shape[0]]
    return (h2, rel)
